# Initial kernel scaffold; baseline (speedup 1.0000x reference)
import functools

import jax
import jax.numpy as jnp
from jax import lax
from jax.experimental import pallas as pl
from jax.experimental.pallas import tpu as pltpu
from jax.experimental.pallas import tpu_sc as plsc

F32 = jnp.float32
_HI = jax.lax.Precision.HIGHEST

_DIMS = [(2, 2, 1, 2), (2, 2, 4, 5), (7, 9, 4, 5), (7, 9, 14, 15),
         (24, 30, 14, 15), (24, 30, 45, 15), (45, 20, 45, 15),
         (45, 20, 35, 10), (30, 3, 35, 2), (30, 3, 5, 2)]

_C = 128
_NW = 32
_WSP = 16


def _dot(a, b):
    return jax.lax.dot_general(a, b, (((1,), (0,)), ((), ())),
                               precision=_HI, preferred_element_type=F32)


def _pad_cols(w, wp):
    return jnp.pad(w, ((0, 0), (0, wp - w.shape[1])))


def _rowmap(fn, nrows, blk, row_ins, consts, out_widths):
    grid = (nrows // blk,)
    in_specs = (
        [pl.BlockSpec((blk, a.shape[1]), lambda g: (g, 0)) for a in row_ins]
        + [pl.BlockSpec(c.shape, lambda g: (0, 0)) for c in consts]
    )
    out_specs = [pl.BlockSpec((blk, w), lambda g: (g, 0)) for w in out_widths]
    out_shape = [jax.ShapeDtypeStruct((nrows, w), F32) for w in out_widths]
    nin = len(row_ins) + len(consts)

    def body(*refs):
        vals = [r[...] for r in refs[:nin]]
        outs = fn(*vals)
        for oref, o in zip(refs[nin:], outs):
            oref[...] = o

    outs = pl.pallas_call(body, grid=grid, in_specs=in_specs,
                          out_specs=out_specs, out_shape=out_shape)(
        *row_ins, *consts)
    return outs


def _sc_gather(table, src, dst):
    _, wp = table.shape
    e = src.shape[0]
    nch = e // _C
    base_tr, extra = nch // _NW, nch % _NW
    mesh = plsc.VectorSubcoreMesh(core_axis_name="c", subcore_axis_name="s")
    out_t = (jax.ShapeDtypeStruct((e, wp), F32),
             jax.ShapeDtypeStruct((e, wp), F32))

    @functools.partial(
        pl.kernel, out_type=out_t, mesh=mesh,
        scratch_types=[pltpu.VMEM((_C,), jnp.int32),
                       pltpu.VMEM((_C,), jnp.int32),
                       pltpu.VMEM((_C, wp), F32),
                       pltpu.VMEM((_C, wp), F32),
                       pltpu.SemaphoreType.DMA,
                       pltpu.SemaphoreType.DMA])
    def k(tab_h, src_h, dst_h, os_h, od_h, si, di, sb, db, sem0, sem1):
        cid = lax.axis_index("c")
        sid = lax.axis_index("s")
        wid = sid * 2 + cid
        trips = base_tr + (wid < extra).astype(jnp.int32)

        def body(j, c):
            off = (wid + _NW * j) * _C
            pltpu.sync_copy(src_h.at[pl.ds(off, _C)], si)
            pltpu.sync_copy(dst_h.at[pl.ds(off, _C)], di)
            c0 = pltpu.async_copy(tab_h.at[si], sb, sem0)
            c1 = pltpu.async_copy(tab_h.at[di], db, sem1)
            c0.wait()
            c1.wait()
            pltpu.sync_copy(sb, os_h.at[pl.ds(off, _C)])
            pltpu.sync_copy(db, od_h.at[pl.ds(off, _C)])
            return c

        lax.fori_loop(0, trips, body, 0)

    return k(table, src, dst)


def _sc_scatter(p, dst, n):
    e, wp = p.shape
    nch = e // _C
    base_tr, extra = nch // _NW, nch % _NW
    rpt = n // 16
    zr = 125
    nz = rpt // zr
    mesh = plsc.VectorSubcoreMesh(core_axis_name="c", subcore_axis_name="s")

    @functools.partial(
        pl.kernel, out_type=jax.ShapeDtypeStruct((2, n, wp), F32), mesh=mesh,
        scratch_types=[pltpu.VMEM((_C,), jnp.int32),
                       pltpu.VMEM((_C, wp), F32),
                       pltpu.VMEM((zr, wp), F32),
                       pltpu.VMEM_SHARED((n, wp), F32)])
    def k(p_h, dst_h, out_h, idx_v, buf_v, zb_v, acc_s):
        cid = lax.axis_index("c")
        sid = lax.axis_index("s")
        wid = sid * 2 + cid
        row0 = sid * rpt

        def zb_body(i, c):
            zb_v[i, :] = jnp.zeros((wp,), F32)
            return c

        lax.fori_loop(0, zr, zb_body, 0)

        def z_body(kk, c):
            pltpu.sync_copy(zb_v, acc_s.at[pl.ds(row0 + kk * zr, zr)])
            return c

        lax.fori_loop(0, nz, z_body, 0)
        plsc.subcore_barrier()

        trips = base_tr + (wid < extra).astype(jnp.int32)

        def body(j, c):
            off = (wid + _NW * j) * _C
            pltpu.sync_copy(dst_h.at[pl.ds(off, _C)], idx_v)
            pltpu.sync_copy(p_h.at[pl.ds(off, _C)], buf_v)
            pltpu.sync_copy(buf_v, acc_s.at[idx_v], add=True)
            return c

        lax.fori_loop(0, trips, body, 0)
        plsc.subcore_barrier()
        pltpu.sync_copy(acc_s.at[pl.ds(row0, rpt)],
                        out_h.at[cid, pl.ds(row0, rpt)])

    return k(p, dst)


def kernel(x, edge_attr, edge_index, params):
    n = x.shape[0]
    e = edge_attr.shape[0]
    src = edge_index[0]
    dst = edge_index[1]
    bn, be = 2500, 2000

    x_carry = ("raw", x)
    ea_carry = ("raw", edge_attr)

    for i in range(5):
        ix, ox, ie, oe = _DIMS[2 * i]
        ixn, oxn, ien, oen = _DIMS[2 * i + 1]
        Wxe, bxe, Wee, bee = params[2 * i]
        Wxn, bxn, Wen, ben = params[2 * i + 1]
        bxe = bxe.reshape(1, -1)
        bee = bee.reshape(1, -1)
        bxn = bxn.reshape(1, -1)
        ben = ben.reshape(1, -1)
        ymode = ox <= ix
        w = ox if ymode else ix
        wp = 16 if w <= 16 else 32

        if x_carry[0] == "raw":
            row_ins = [x_carry[1]]
            consts = []
            oen_p = None
        else:
            _, x2p, p0, p1, benp = x_carry
            row_ins = [x2p, p0, p1]
            consts = [benp]
            oen_p = benp.shape[1]
        if ymode:
            consts = consts + [_pad_cols(Wxe, wp)]
        consts = consts + [Wxn, bxn]

        def nk_fn(*vals, _ymode=ymode, _wp=wp, _oen_p=oen_p,
                  _raw=(x_carry[0] == "raw")):
            vals = list(vals)
            if _raw:
                xf = vals.pop(0)
            else:
                x2p_b, p0_b, p1_b, benp_b = vals[0], vals[1], vals[2], vals[3]
                vals = vals[4:]
                xa = jnp.maximum(x2p_b, 0.0)
                xb = jnp.maximum((p0_b + p1_b)[:, :_oen_p] + benp_b, 0.0)
                xf = jnp.concatenate([xa, xb], axis=1)
            if _ymode:
                wxe_p = vals.pop(0)
                table = _dot(xf, wxe_p)
            else:
                table = jnp.pad(xf, ((0, 0), (0, _wp - xf.shape[1])))
            wxn_b, bxn_b = vals[0], vals[1]
            x2 = _dot(xf, wxn_b) + bxn_b
            return [table, x2]

        table, x2 = _rowmap(nk_fn, n, bn, row_ins, consts, [wp, oxn])

        gs, gd = _sc_gather(table, src, dst)

        if ea_carry[0] == "raw":
            e_row_ins = [ea_carry[1], gs, gd]
            wee_parts = [Wee]
        else:
            _, a_prev, b_prev = ea_carry
            e_row_ins = [a_prev, b_prev, gs, gd]
            sp = a_prev.shape[1]
            wee_parts = [Wee[:sp], Wee[sp:]]
        e_consts = list(wee_parts) + [bee]
        if not ymode:
            e_consts.append(jnp.pad(Wxe, ((0, wp - ix), (0, 0))))
        e_consts.append(bxe)
        e_consts.append(_pad_cols(Wen[:oe], _WSP))
        e_consts.append(_pad_cols(Wen[oe:], _WSP))
        last = i == 4
        n_ea = 1 if ea_carry[0] == "raw" else 2

        def ek_fn(*vals, _ymode=ymode, _ox=ox, _n_ea=n_ea, _last=last):
            vals = list(vals)
            ea_parts = [vals.pop(0) for _ in range(_n_ea)]
            gs_b = vals.pop(0)
            gd_b = vals.pop(0)
            wee_p = [vals.pop(0) for _ in range(_n_ea)]
            bee_b = vals.pop(0)
            ea2 = sum(_dot(ap, wp_) for ap, wp_ in zip(ea_parts, wee_p)) + bee_b
            g = gs_b + gd_b
            if _ymode:
                ns = g[:, :_ox] + vals.pop(0)
            else:
                wxe_b = vals.pop(0)
                ns = _dot(g, wxe_b) + vals.pop(0)
            a_ = jnp.maximum(ea2, 0.0)
            b_ = jnp.maximum(ns, 0.0)
            p_ = _dot(a_, vals.pop(0)) + _dot(b_, vals.pop(0))
            if _last:
                return [p_]
            return [a_, b_, p_]

        out_w = [_WSP] if last else [oe, ox, _WSP]
        ek_outs = _rowmap(ek_fn, e, be, e_row_ins, e_consts, out_w)
        if last:
            p_pay = ek_outs[0]
            ea_carry = None
        else:
            a_new, b_new, p_pay = ek_outs
            ea_carry = ("parts", a_new, b_new)

        partials = _sc_scatter(p_pay, dst, n)
        x_carry = ("pend", x2, partials[0], partials[1], ben)

    _, x2f, p0f, p1f, benf = x_carry
    oenf = benf.shape[1]

    grid = (n // bn,)

    def f1_body(x2_r, p0_r, p1_r, ben_r, o5_r, s_r):
        xa = jnp.maximum(x2_r[...], 0.0)
        xb = jnp.maximum((p0_r[...] + p1_r[...])[:, :oenf] + ben_r[...], 0.0)
        x5 = jnp.concatenate([xa, xb], axis=1)
        o5_r[...] = x5

        @pl.when(pl.program_id(0) == 0)
        def _():
            s_r[...] = jnp.zeros_like(s_r)

        s_r[...] += jnp.sum(x5, axis=0, keepdims=True)

    d5 = x2f.shape[1] + oenf
    o5, ssum = pl.pallas_call(
        f1_body, grid=grid,
        in_specs=[pl.BlockSpec((bn, x2f.shape[1]), lambda g: (g, 0)),
                  pl.BlockSpec((bn, p0f.shape[1]), lambda g: (g, 0)),
                  pl.BlockSpec((bn, p1f.shape[1]), lambda g: (g, 0)),
                  pl.BlockSpec(benf.shape, lambda g: (0, 0))],
        out_specs=[pl.BlockSpec((bn, d5), lambda g: (g, 0)),
                   pl.BlockSpec((1, d5), lambda g: (0, 0))],
        out_shape=[jax.ShapeDtypeStruct((n, d5), F32),
                   jax.ShapeDtypeStruct((1, d5), F32)],
    )(x2f, p0f, p1f, benf)

    def f2_body(o5_r, s_r, out_r):
        out_r[...] = jnp.concatenate(
            [o5_r[...], jnp.broadcast_to(s_r[...], (bn, d5))], axis=1)

    out = pl.pallas_call(
        f2_body, grid=grid,
        in_specs=[pl.BlockSpec((bn, d5), lambda g: (g, 0)),
                  pl.BlockSpec((1, d5), lambda g: (0, 0))],
        out_specs=pl.BlockSpec((bn, 2 * d5), lambda g: (g, 0)),
        out_shape=jax.ShapeDtypeStruct((n, 2 * d5), F32),
    )(o5, ssum)
    return out


# speedup vs baseline: 1.4102x; 1.4102x over previous
import functools

import jax
import jax.numpy as jnp
from jax import lax
from jax.experimental import pallas as pl
from jax.experimental.pallas import tpu as pltpu
from jax.experimental.pallas import tpu_sc as plsc

F32 = jnp.float32
_HI = jax.lax.Precision.HIGHEST

_DIMS = [(2, 2, 1, 2), (2, 2, 4, 5), (7, 9, 4, 5), (7, 9, 14, 15),
         (24, 30, 14, 15), (24, 30, 45, 15), (45, 20, 45, 15),
         (45, 20, 35, 10), (30, 3, 35, 2), (30, 3, 5, 2)]

_C = 128
_NW = 32
_WSP = 16


def _dot(a, b):
    return jax.lax.dot_general(a, b, (((1,), (0,)), ((), ())),
                               precision=_HI, preferred_element_type=F32)


def _pad_cols(w, wp):
    return jnp.pad(w, ((0, 0), (0, wp - w.shape[1])))


def _rowmap(fn, nrows, blk, row_ins, consts, out_widths):
    grid = (nrows // blk,)
    in_specs = (
        [pl.BlockSpec((blk, a.shape[1]), lambda g: (g, 0)) for a in row_ins]
        + [pl.BlockSpec(c.shape, lambda g: (0, 0)) for c in consts]
    )
    out_specs = [pl.BlockSpec((blk, w), lambda g: (g, 0)) for w in out_widths]
    out_shape = [jax.ShapeDtypeStruct((nrows, w), F32) for w in out_widths]
    nin = len(row_ins) + len(consts)

    def body(*refs):
        vals = [r[...] for r in refs[:nin]]
        outs = fn(*vals)
        for oref, o in zip(refs[nin:], outs):
            oref[...] = o

    outs = pl.pallas_call(body, grid=grid, in_specs=in_specs,
                          out_specs=out_specs, out_shape=out_shape)(
        *row_ins, *consts)
    return outs


def _sc_gather(table, src, dst):
    _, wp = table.shape
    e = src.shape[0]
    nch = e // _C
    base_tr, extra = nch // _NW, nch % _NW
    mesh = plsc.VectorSubcoreMesh(core_axis_name="c", subcore_axis_name="s")
    out_t = (jax.ShapeDtypeStruct((e, wp), F32),
             jax.ShapeDtypeStruct((e, wp), F32))

    @functools.partial(
        pl.kernel, out_type=out_t, mesh=mesh,
        compiler_params=pltpu.CompilerParams(use_tc_tiling_on_sc=False),
        scratch_types=[pltpu.VMEM((_C,), jnp.int32),
                       pltpu.VMEM((_C,), jnp.int32),
                       pltpu.VMEM((_C, wp), F32),
                       pltpu.VMEM((_C, wp), F32),
                       pltpu.SemaphoreType.DMA,
                       pltpu.SemaphoreType.DMA])
    def k(tab_h, src_h, dst_h, os_h, od_h, si, di, sb, db, sem0, sem1):
        cid = lax.axis_index("c")
        sid = lax.axis_index("s")
        wid = sid * 2 + cid
        trips = base_tr + (wid < extra).astype(jnp.int32)

        def body(j, c):
            off = (wid + _NW * j) * _C
            pltpu.sync_copy(src_h.at[pl.ds(off, _C)], si)
            pltpu.sync_copy(dst_h.at[pl.ds(off, _C)], di)
            c0 = pltpu.async_copy(tab_h.at[si], sb, sem0)
            c1 = pltpu.async_copy(tab_h.at[di], db, sem1)
            c0.wait()
            c1.wait()
            pltpu.sync_copy(sb, os_h.at[pl.ds(off, _C)])
            pltpu.sync_copy(db, od_h.at[pl.ds(off, _C)])
            return c

        lax.fori_loop(0, trips, body, 0)

    return k(table, src, dst)


def _sc_scatter(p, dst, n):
    e, wp = p.shape
    nch = e // _C
    base_tr, extra = nch // _NW, nch % _NW
    rpt = n // 16
    zr = 125
    nz = rpt // zr
    mesh = plsc.VectorSubcoreMesh(core_axis_name="c", subcore_axis_name="s")

    @functools.partial(
        pl.kernel, out_type=jax.ShapeDtypeStruct((2, n, wp), F32), mesh=mesh,
        compiler_params=pltpu.CompilerParams(use_tc_tiling_on_sc=False),
        scratch_types=[pltpu.VMEM((_C,), jnp.int32),
                       pltpu.VMEM((_C, wp), F32),
                       pltpu.VMEM((zr, wp), F32),
                       pltpu.VMEM_SHARED((n, wp), F32)])
    def k(p_h, dst_h, out_h, idx_v, buf_v, zb_v, acc_s):
        cid = lax.axis_index("c")
        sid = lax.axis_index("s")
        wid = sid * 2 + cid
        row0 = sid * rpt

        def zb_body(i, c):
            zb_v[i, :] = jnp.zeros((wp,), F32)
            return c

        lax.fori_loop(0, zr, zb_body, 0)

        def z_body(kk, c):
            pltpu.sync_copy(zb_v, acc_s.at[pl.ds(row0 + kk * zr, zr)])
            return c

        lax.fori_loop(0, nz, z_body, 0)
        plsc.subcore_barrier()

        trips = base_tr + (wid < extra).astype(jnp.int32)

        def body(j, c):
            off = (wid + _NW * j) * _C
            pltpu.sync_copy(dst_h.at[pl.ds(off, _C)], idx_v)
            pltpu.sync_copy(p_h.at[pl.ds(off, _C)], buf_v)
            pltpu.sync_copy(buf_v, acc_s.at[idx_v], add=True)
            return c

        lax.fori_loop(0, trips, body, 0)
        plsc.subcore_barrier()
        pltpu.sync_copy(acc_s.at[pl.ds(row0, rpt)],
                        out_h.at[cid, pl.ds(row0, rpt)])

    return k(p, dst)


def kernel(x, edge_attr, edge_index, params):
    n = x.shape[0]
    e = edge_attr.shape[0]
    src = edge_index[0]
    dst = edge_index[1]
    bn, be = 2000, 2000

    x_carry = ("raw", x)
    ea_carry = ("raw", edge_attr)

    for i in range(5):
        ix, ox, ie, oe = _DIMS[2 * i]
        ixn, oxn, ien, oen = _DIMS[2 * i + 1]
        Wxe, bxe, Wee, bee = params[2 * i]
        Wxn, bxn, Wen, ben = params[2 * i + 1]
        bxe = bxe.reshape(1, -1)
        bee = bee.reshape(1, -1)
        bxn = bxn.reshape(1, -1)
        ben = ben.reshape(1, -1)
        ymode = ox <= ix
        w = ox if ymode else ix
        wp = 16 if w <= 16 else 32

        if x_carry[0] == "raw":
            row_ins = [x_carry[1]]
            consts = []
            oen_p = None
        else:
            _, x2p, p0, p1, benp = x_carry
            row_ins = [x2p, p0, p1]
            consts = [benp]
            oen_p = benp.shape[1]
        if ymode:
            consts = consts + [_pad_cols(Wxe, wp)]
        consts = consts + [Wxn, bxn]

        def nk_fn(*vals, _ymode=ymode, _wp=wp, _oen_p=oen_p,
                  _raw=(x_carry[0] == "raw")):
            vals = list(vals)
            if _raw:
                xf = vals.pop(0)
            else:
                x2p_b, p0_b, p1_b, benp_b = vals[0], vals[1], vals[2], vals[3]
                vals = vals[4:]
                xa = jnp.maximum(x2p_b, 0.0)
                xb = jnp.maximum((p0_b + p1_b)[:, :_oen_p] + benp_b, 0.0)
                xf = jnp.concatenate([xa, xb], axis=1)
            if _ymode:
                wxe_p = vals.pop(0)
                table = _dot(xf, wxe_p)
            else:
                table = jnp.pad(xf, ((0, 0), (0, _wp - xf.shape[1])))
            wxn_b, bxn_b = vals[0], vals[1]
            x2 = _dot(xf, wxn_b) + bxn_b
            return [table, x2]

        table, x2 = _rowmap(nk_fn, n, bn, row_ins, consts, [wp, oxn])

        gs, gd = _sc_gather(table, src, dst)

        if ea_carry[0] == "raw":
            e_row_ins = [ea_carry[1], gs, gd]
            wee_parts = [Wee]
        else:
            _, a_prev, b_prev = ea_carry
            e_row_ins = [a_prev, b_prev, gs, gd]
            sp = a_prev.shape[1]
            wee_parts = [Wee[:sp], Wee[sp:]]
        e_consts = list(wee_parts) + [bee]
        if not ymode:
            e_consts.append(jnp.pad(Wxe, ((0, wp - ix), (0, 0))))
        e_consts.append(bxe)
        e_consts.append(_pad_cols(Wen[:oe], _WSP))
        e_consts.append(_pad_cols(Wen[oe:], _WSP))
        last = i == 4
        n_ea = 1 if ea_carry[0] == "raw" else 2

        def ek_fn(*vals, _ymode=ymode, _ox=ox, _n_ea=n_ea, _last=last):
            vals = list(vals)
            ea_parts = [vals.pop(0) for _ in range(_n_ea)]
            gs_b = vals.pop(0)
            gd_b = vals.pop(0)
            wee_p = [vals.pop(0) for _ in range(_n_ea)]
            bee_b = vals.pop(0)
            ea2 = sum(_dot(ap, wp_) for ap, wp_ in zip(ea_parts, wee_p)) + bee_b
            g = gs_b + gd_b
            if _ymode:
                ns = g[:, :_ox] + vals.pop(0)
            else:
                wxe_b = vals.pop(0)
                ns = _dot(g, wxe_b) + vals.pop(0)
            a_ = jnp.maximum(ea2, 0.0)
            b_ = jnp.maximum(ns, 0.0)
            p_ = _dot(a_, vals.pop(0)) + _dot(b_, vals.pop(0))
            if _last:
                return [p_]
            return [a_, b_, p_]

        out_w = [_WSP] if last else [oe, ox, _WSP]
        ek_outs = _rowmap(ek_fn, e, be, e_row_ins, e_consts, out_w)
        if last:
            p_pay = ek_outs[0]
            ea_carry = None
        else:
            a_new, b_new, p_pay = ek_outs
            ea_carry = ("parts", a_new, b_new)

        partials = _sc_scatter(p_pay, dst, n)
        x_carry = ("pend", x2, partials[0], partials[1], ben)

    _, x2f, p0f, p1f, benf = x_carry
    oenf = benf.shape[1]

    grid = (n // bn,)

    def f1_body(x2_r, p0_r, p1_r, ben_r, o5_r, s_r):
        xa = jnp.maximum(x2_r[...], 0.0)
        xb = jnp.maximum((p0_r[...] + p1_r[...])[:, :oenf] + ben_r[...], 0.0)
        x5 = jnp.concatenate([xa, xb], axis=1)
        o5_r[...] = x5

        @pl.when(pl.program_id(0) == 0)
        def _():
            s_r[...] = jnp.zeros_like(s_r)

        s_r[...] += jnp.sum(x5, axis=0, keepdims=True)

    d5 = x2f.shape[1] + oenf
    o5, ssum = pl.pallas_call(
        f1_body, grid=grid,
        in_specs=[pl.BlockSpec((bn, x2f.shape[1]), lambda g: (g, 0)),
                  pl.BlockSpec((bn, p0f.shape[1]), lambda g: (g, 0)),
                  pl.BlockSpec((bn, p1f.shape[1]), lambda g: (g, 0)),
                  pl.BlockSpec(benf.shape, lambda g: (0, 0))],
        out_specs=[pl.BlockSpec((bn, d5), lambda g: (g, 0)),
                   pl.BlockSpec((1, d5), lambda g: (0, 0))],
        out_shape=[jax.ShapeDtypeStruct((n, d5), F32),
                   jax.ShapeDtypeStruct((1, d5), F32)],
    )(x2f, p0f, p1f, benf)

    def f2_body(o5_r, s_r, out_r):
        out_r[...] = jnp.concatenate(
            [o5_r[...], jnp.broadcast_to(s_r[...], (bn, d5))], axis=1)

    out = pl.pallas_call(
        f2_body, grid=grid,
        in_specs=[pl.BlockSpec((bn, d5), lambda g: (g, 0)),
                  pl.BlockSpec((1, d5), lambda g: (0, 0))],
        out_specs=pl.BlockSpec((bn, 2 * d5), lambda g: (g, 0)),
        out_shape=jax.ShapeDtypeStruct((n, 2 * d5), F32),
    )(o5, ssum)
    return out


# speedup vs baseline: 3.5622x; 2.5260x over previous
import functools

import jax
import jax.numpy as jnp
from jax import lax
from jax.experimental import pallas as pl
from jax.experimental.pallas import tpu as pltpu
from jax.experimental.pallas import tpu_sc as plsc

F32 = jnp.float32
_HI = jax.lax.Precision.HIGHEST

_DIMS = [(2, 2, 1, 2), (2, 2, 4, 5), (7, 9, 4, 5), (7, 9, 14, 15),
         (24, 30, 14, 15), (24, 30, 45, 15), (45, 20, 45, 15),
         (45, 20, 35, 10), (30, 3, 35, 2), (30, 3, 5, 2)]

_C = 128
_NW = 32
_WSP = 16


def _dot(a, b):
    return jax.lax.dot_general(a, b, (((1,), (0,)), ((), ())),
                               precision=_HI, preferred_element_type=F32)


def _pad_cols(w, wp):
    return jnp.pad(w, ((0, 0), (0, wp - w.shape[1])))


def _rowmap(fn, nrows, blk, row_ins, consts, out_widths):
    grid = (nrows // blk,)
    in_specs = (
        [pl.BlockSpec((blk, a.shape[1]), lambda g: (g, 0)) for a in row_ins]
        + [pl.BlockSpec(c.shape, lambda g: (0, 0)) for c in consts]
    )
    out_specs = [pl.BlockSpec((blk, w), lambda g: (g, 0)) for w in out_widths]
    out_shape = [jax.ShapeDtypeStruct((nrows, w), F32) for w in out_widths]
    nin = len(row_ins) + len(consts)

    def body(*refs):
        vals = [r[...] for r in refs[:nin]]
        outs = fn(*vals)
        for oref, o in zip(refs[nin:], outs):
            oref[...] = o

    outs = pl.pallas_call(body, grid=grid, in_specs=in_specs,
                          out_specs=out_specs, out_shape=out_shape)(
        *row_ins, *consts)
    return outs


def _sc_gather(tables, src, dst):
    ntab = len(tables)
    e = src.shape[0]
    nch = e // _C
    base_tr, extra = nch // _NW, nch % _NW
    mesh = plsc.VectorSubcoreMesh(core_axis_name="c", subcore_axis_name="s")
    out_t = tuple(jax.ShapeDtypeStruct((e, 16), F32) for _ in range(2 * ntab))

    @functools.partial(
        pl.kernel, out_type=out_t, mesh=mesh,
        compiler_params=pltpu.CompilerParams(use_tc_tiling_on_sc=False),
        scratch_types=([pltpu.VMEM((_C,), jnp.int32)] * 2
                       + [pltpu.VMEM((_C, 16), F32)] * (2 * ntab)
                       + [pltpu.SemaphoreType.DMA] * (2 * ntab)))
    def k(*refs):
        tabs = refs[:ntab]
        src_h, dst_h = refs[ntab], refs[ntab + 1]
        outs = refs[ntab + 2:ntab + 2 + 2 * ntab]
        si, di = refs[ntab + 2 + 2 * ntab], refs[ntab + 3 + 2 * ntab]
        bufs = refs[ntab + 4 + 2 * ntab:ntab + 4 + 4 * ntab]
        sems = refs[ntab + 4 + 4 * ntab:]
        cid = lax.axis_index("c")
        sid = lax.axis_index("s")
        wid = sid * 2 + cid
        trips = base_tr + (wid < extra).astype(jnp.int32)

        def body(j, c):
            off = (wid + _NW * j) * _C
            pltpu.sync_copy(src_h.at[pl.ds(off, _C)], si)
            pltpu.sync_copy(dst_h.at[pl.ds(off, _C)], di)
            cps = []
            for t in range(ntab):
                cps.append(pltpu.async_copy(tabs[t].at[si], bufs[2 * t],
                                            sems[2 * t]))
                cps.append(pltpu.async_copy(tabs[t].at[di], bufs[2 * t + 1],
                                            sems[2 * t + 1]))
            for cp in cps:
                cp.wait()
            for t in range(2 * ntab):
                pltpu.sync_copy(bufs[t], outs[t].at[pl.ds(off, _C)])
            return c

        lax.fori_loop(0, trips, body, 0)

    return k(*tables, src, dst)


def _sc_scatter(p, dst, n):
    e, wp = p.shape
    nch = e // _C
    base_tr, extra = nch // _NW, nch % _NW
    rpt = n // 16
    zr = 128
    nz = rpt // zr
    mesh = plsc.VectorSubcoreMesh(core_axis_name="c", subcore_axis_name="s")

    @functools.partial(
        pl.kernel,
        out_type=(jax.ShapeDtypeStruct((n, wp), F32),
                  jax.ShapeDtypeStruct((n, wp), F32)),
        mesh=mesh,
        compiler_params=pltpu.CompilerParams(use_tc_tiling_on_sc=False),
        scratch_types=[pltpu.VMEM((_C,), jnp.int32),
                       pltpu.VMEM((_C, wp), F32),
                       pltpu.VMEM((zr, wp), F32),
                       pltpu.VMEM_SHARED((n, wp), F32)])
    def k(p_h, dst_h, o0_h, o1_h, idx_v, buf_v, zb_v, acc_s):
        cid = lax.axis_index("c")
        sid = lax.axis_index("s")
        wid = sid * 2 + cid
        row0 = sid * rpt

        def zb_body(i, c):
            zb_v[i, :] = jnp.zeros((wp,), F32)
            return c

        lax.fori_loop(0, zr, zb_body, 0)

        def z_body(kk, c):
            pltpu.sync_copy(zb_v, acc_s.at[pl.ds(row0 + kk * zr, zr)])
            return c

        lax.fori_loop(0, nz, z_body, 0)
        plsc.subcore_barrier()

        trips = base_tr + (wid < extra).astype(jnp.int32)

        def body(j, c):
            off = (wid + _NW * j) * _C
            pltpu.sync_copy(dst_h.at[pl.ds(off, _C)], idx_v)
            pltpu.sync_copy(p_h.at[pl.ds(off, _C)], buf_v)
            pltpu.sync_copy(buf_v, acc_s.at[idx_v], add=True)
            return c

        lax.fori_loop(0, trips, body, 0)
        plsc.subcore_barrier()

        @pl.when(cid == 0)
        def _():
            pltpu.sync_copy(acc_s.at[pl.ds(row0, rpt)],
                            o0_h.at[pl.ds(row0, rpt)])

        @pl.when(cid == 1)
        def _():
            pltpu.sync_copy(acc_s.at[pl.ds(row0, rpt)],
                            o1_h.at[pl.ds(row0, rpt)])

    return k(p, dst)


def _kron8(w):
    return jnp.kron(jnp.eye(8, dtype=F32), w)


def _tile8(b):
    return jnp.tile(b.reshape(1, -1), (1, 8))


def _eye(r, c, off=0):
    return jnp.eye(r, c, off, dtype=F32)


def kernel(x, edge_attr, edge_index, params):
    n = x.shape[0]
    e = edge_attr.shape[0]
    npad = 51200
    n8 = npad // 8
    e8 = e // 8
    src = edge_index[0]
    dst = edge_index[1]
    be = 1000
    bnp = 800

    xp = jnp.pad(x, ((0, npad - n), (0, 0))).reshape(n8, 8 * x.shape[1])
    eap = edge_attr.reshape(e8, 8 * edge_attr.shape[1])

    x_carry = ("raw", xp)
    ea_carry = ("raw", eap)

    for i in range(5):
        ix, ox, ie, oe = _DIMS[2 * i]
        ixn, oxn, ien, oen = _DIMS[2 * i + 1]
        Wxe, bxe, Wee, bee = params[2 * i]
        Wxn, bxn, Wen, ben = params[2 * i + 1]
        ymode = ox <= ix
        w = ox if ymode else ix
        nhalf = 1 if w <= 16 else 2

        if x_carry[0] == "raw":
            row_ins = [x_carry[1]]
            pre = []
            n_parts = 1
            ixn_prev = x.shape[1]
            splits = [(0, ixn_prev)]
        else:
            _, x2p, p0, p1, ben_prev, oen_prev = x_carry
            oxn_prev = x2p.shape[1] // 8
            row_ins = [x2p, p0, p1]
            pre = [_kron8(_eye(_WSP, oen_prev)), _tile8(ben_prev)]
            n_parts = 2
            splits = [(0, oxn_prev), (oxn_prev, oxn_prev + oen_prev)]
        t_ws = []
        for t in range(nhalf):
            for (lo, hi) in splits:
                if ymode:
                    t_ws.append(_kron8(_pad_cols(
                        Wxe[lo:hi, 16 * t:16 * (t + 1)], 16)))
                else:
                    t_ws.append(_kron8(_eye(hi - lo, 16, lo - 16 * t)))
        n_ws = [_kron8(Wxn[lo:hi]) for (lo, hi) in splits]
        consts = pre + t_ws + n_ws + [_tile8(bxn)]

        def nk_fn(*vals, _np=n_parts, _nh=nhalf):
            vals = list(vals)
            if _np == 1:
                parts = [vals.pop(0)]
            else:
                x2p_b, p0_b, p1_b = vals[0], vals[1], vals[2]
                sel_b, bent_b = vals[3], vals[4]
                vals = vals[5:]
                xa = jnp.maximum(x2p_b, 0.0)
                xb = jnp.maximum(_dot(p0_b + p1_b, sel_b) + bent_b, 0.0)
                parts = [xa, xb]
            tw = [vals.pop(0) for _ in range(_nh * _np)]
            nw = [vals.pop(0) for _ in range(_np)]
            bxn_b = vals.pop(0)
            tables = [sum(_dot(p_, tw[t * _np + j])
                          for j, p_ in enumerate(parts))
                      for t in range(_nh)]
            x2 = sum(_dot(p_, w_) for p_, w_ in zip(parts, nw)) + bxn_b
            return tables + [x2]

        nk_outs = _rowmap(nk_fn, n8, bnp, row_ins, consts,
                          [128] * nhalf + [8 * oxn])
        tables = [tp.reshape(npad, 16) for tp in nk_outs[:nhalf]]
        x2 = nk_outs[nhalf]

        g_outs = _sc_gather(tables, src, dst)
        g_packed = [o.reshape(e8, 128) for o in g_outs]

        last = i == 4
        if ea_carry[0] == "raw":
            ea_ins = [ea_carry[1]]
            wee_parts = [_kron8(Wee)]
        else:
            _, a_prev, b_prev = ea_carry
            ea_ins = [a_prev, b_prev]
            sp = a_prev.shape[1] // 8
            wee_parts = [_kron8(Wee[:sp]), _kron8(Wee[sp:])]
        wx_half = []
        for t in range(nhalf):
            if ymode:
                wx_half.append(_kron8(_eye(16, ox, 16 * t)))
            else:
                blkw = Wxe[16 * t:16 * (t + 1)]
                wx_half.append(_kron8(jnp.pad(
                    blkw, ((0, 16 - blkw.shape[0]), (0, 0)))))
        e_consts = (wee_parts + [_tile8(bee)] + wx_half
                    + [_tile8(bxe),
                       _kron8(_pad_cols(Wen[:oe], _WSP)),
                       _kron8(_pad_cols(Wen[oe:], _WSP))])
        n_ea = len(ea_ins)

        def ek_fn(*vals, _n_ea=n_ea, _last=last, _nh=nhalf):
            vals = list(vals)
            ea_parts = [vals.pop(0) for _ in range(_n_ea)]
            g_bs = [vals.pop(0) for _ in range(2 * _nh)]
            wee_p = [vals.pop(0) for _ in range(_n_ea)]
            bee_b = vals.pop(0)
            wx_b = [vals.pop(0) for _ in range(_nh)]
            bxe_b = vals.pop(0)
            wna_b = vals.pop(0)
            wnb_b = vals.pop(0)
            ea2 = sum(_dot(ap, w_) for ap, w_ in zip(ea_parts, wee_p)) + bee_b
            ns = sum(_dot(g_bs[2 * t] + g_bs[2 * t + 1], wx_b[t])
                     for t in range(_nh)) + bxe_b
            a_ = jnp.maximum(ea2, 0.0)
            b_ = jnp.maximum(ns, 0.0)
            p_ = _dot(a_, wna_b) + _dot(b_, wnb_b)
            if _last:
                return [p_]
            return [a_, b_, p_]

        out_w = [8 * _WSP] if last else [8 * oe, 8 * ox, 8 * _WSP]
        ek_outs = _rowmap(ek_fn, e8, be, ea_ins + g_packed, e_consts, out_w)
        if last:
            p_pay = ek_outs[0]
            ea_carry = None
        else:
            a_new, b_new, p_pay = ek_outs
            ea_carry = ("parts", a_new, b_new)

        part0, part1 = _sc_scatter(p_pay.reshape(e, _WSP), dst, npad)
        p0 = part0.reshape(n8, 8 * _WSP)
        p1 = part1.reshape(n8, 8 * _WSP)
        x_carry = ("pend", x2, p0, p1, ben, oen)

    _, x2f, p0f, p1f, benf, oenf = x_carry
    oxnf = x2f.shape[1] // 8
    d5 = oxnf + oenf
    sel_es = _kron8(_eye(_WSP, oenf))
    ben_t = _tile8(benf)
    ka = _kron8(_eye(oxnf, d5))
    kb = _kron8(_eye(oenf, d5, oxnf))
    gsum = jnp.tile(_eye(d5, d5), (8, 1))
    kx = _kron8(_eye(d5, 2 * d5))
    ks = jnp.tile(_eye(d5, 2 * d5, d5), (1, 8))
    n8_valid = n // 8

    grid = (n8 // bnp,)

    def f1_body(x2_r, p0_r, p1_r, sel_r, bent_r, ka_r, kb_r, o5_r, s_r):
        xa = jnp.maximum(x2_r[...], 0.0)
        xb = jnp.maximum(_dot(p0_r[...] + p1_r[...], sel_r[...])
                         + bent_r[...], 0.0)
        x5 = _dot(xa, ka_r[...]) + _dot(xb, kb_r[...])
        o5_r[...] = x5
        rid = (pl.program_id(0) * bnp
               + jax.lax.broadcasted_iota(jnp.int32, (bnp, 1), 0))
        x5m = jnp.where(rid < n8_valid, x5, 0.0)

        @pl.when(pl.program_id(0) == 0)
        def _():
            s_r[...] = jnp.zeros_like(s_r)

        s_r[...] += jnp.sum(x5m, axis=0, keepdims=True)

    o5, ssum = pl.pallas_call(
        f1_body, grid=grid,
        in_specs=[pl.BlockSpec((bnp, x2f.shape[1]), lambda g: (g, 0)),
                  pl.BlockSpec((bnp, 8 * _WSP), lambda g: (g, 0)),
                  pl.BlockSpec((bnp, 8 * _WSP), lambda g: (g, 0)),
                  pl.BlockSpec(sel_es.shape, lambda g: (0, 0)),
                  pl.BlockSpec(ben_t.shape, lambda g: (0, 0)),
                  pl.BlockSpec(ka.shape, lambda g: (0, 0)),
                  pl.BlockSpec(kb.shape, lambda g: (0, 0))],
        out_specs=[pl.BlockSpec((bnp, 8 * d5), lambda g: (g, 0)),
                   pl.BlockSpec((1, 8 * d5), lambda g: (0, 0))],
        out_shape=[jax.ShapeDtypeStruct((n8, 8 * d5), F32),
                   jax.ShapeDtypeStruct((1, 8 * d5), F32)],
    )(x2f, p0f, p1f, sel_es, ben_t, ka, kb)

    def f2_body(o5_r, s_r, gs_r, kx_r, ks_r, out_r):
        s5 = _dot(s_r[...], gs_r[...])
        out_r[...] = _dot(o5_r[...], kx_r[...]) + jnp.broadcast_to(
            _dot(s5, ks_r[...]), (bnp, 16 * d5))

    outp = pl.pallas_call(
        f2_body, grid=grid,
        in_specs=[pl.BlockSpec((bnp, 8 * d5), lambda g: (g, 0)),
                  pl.BlockSpec((1, 8 * d5), lambda g: (0, 0)),
                  pl.BlockSpec(gsum.shape, lambda g: (0, 0)),
                  pl.BlockSpec(kx.shape, lambda g: (0, 0)),
                  pl.BlockSpec(ks.shape, lambda g: (0, 0))],
        out_specs=pl.BlockSpec((bnp, 16 * d5), lambda g: (g, 0)),
        out_shape=jax.ShapeDtypeStruct((n8, 16 * d5), F32),
    )(o5, ssum, gsum, kx, ks)
    return outp.reshape(npad, 2 * d5)[:n]


# speedup vs baseline: 4.9559x; 1.3912x over previous
import functools

import jax
import jax.numpy as jnp
from jax import lax
from jax.experimental import pallas as pl
from jax.experimental.pallas import tpu as pltpu
from jax.experimental.pallas import tpu_sc as plsc

F32 = jnp.float32
_HI = jax.lax.Precision.HIGHEST

_DIMS = [(2, 2, 1, 2), (2, 2, 4, 5), (7, 9, 4, 5), (7, 9, 14, 15),
         (24, 30, 14, 15), (24, 30, 45, 15), (45, 20, 45, 15),
         (45, 20, 35, 10), (30, 3, 35, 2), (30, 3, 5, 2)]

_C = 128
_NW = 32
_WSP = 16


def _dot(a, b):
    return jax.lax.dot_general(a, b, (((1,), (0,)), ((), ())),
                               precision=_HI, preferred_element_type=F32)


def _pad_cols(w, wp):
    return jnp.pad(w, ((0, 0), (0, wp - w.shape[1])))


def _rowmap(fn, nrows, blk, row_ins, consts, out_widths):
    grid = (nrows // blk,)
    in_specs = (
        [pl.BlockSpec((blk, a.shape[1]), lambda g: (g, 0)) for a in row_ins]
        + [pl.BlockSpec(c.shape, lambda g: (0, 0)) for c in consts]
    )
    out_specs = [pl.BlockSpec((blk, w), lambda g: (g, 0)) for w in out_widths]
    out_shape = [jax.ShapeDtypeStruct((nrows, w), F32) for w in out_widths]
    nin = len(row_ins) + len(consts)

    def body(*refs):
        vals = [r[...] for r in refs[:nin]]
        outs = fn(*vals)
        for oref, o in zip(refs[nin:], outs):
            oref[...] = o

    outs = pl.pallas_call(body, grid=grid, in_specs=in_specs,
                          out_specs=out_specs, out_shape=out_shape)(
        *row_ins, *consts)
    return outs


def _sc_gather(tables, src, dst):
    ntab = len(tables)
    e = src.shape[0]
    nch = e // _C
    base_tr, extra = nch // _NW, nch % _NW
    mesh = plsc.VectorSubcoreMesh(core_axis_name="c", subcore_axis_name="s")
    out_t = tuple(jax.ShapeDtypeStruct((e, 16), F32) for _ in range(2 * ntab))

    l2 = 2 * ntab

    @functools.partial(
        pl.kernel, out_type=out_t, mesh=mesh,
        compiler_params=pltpu.CompilerParams(use_tc_tiling_on_sc=False),
        scratch_types=([pltpu.VMEM((_C,), jnp.int32)] * 4
                       + [pltpu.VMEM((_C, 16), F32)] * (2 * l2)
                       + [pltpu.SemaphoreType.DMA] * 6))
    def k(*refs):
        tabs = refs[:ntab]
        src_h, dst_h = refs[ntab], refs[ntab + 1]
        outs = refs[ntab + 2:ntab + 2 + l2]
        sc = list(refs[ntab + 2 + l2:])
        si = [sc[0], sc[2]]
        di = [sc[1], sc[3]]
        bufs = [sc[4:4 + l2], sc[4 + l2:4 + 2 * l2]]
        semi = [sc[4 + 2 * l2], sc[5 + 2 * l2]]
        semg = [sc[6 + 2 * l2], sc[7 + 2 * l2]]
        semw = [sc[8 + 2 * l2], sc[9 + 2 * l2]]
        cid = lax.axis_index("c")
        sid = lax.axis_index("s")
        wid = sid * 2 + cid
        trips = base_tr + (wid < extra).astype(jnp.int32)
        pairs = trips // 2
        odd = trips - 2 * pairs

        def off_of(c):
            return (wid + _NW * c) * _C

        def issue_idx(c, s):
            off = off_of(c)
            pltpu.async_copy(src_h.at[pl.ds(off, _C)], si[s], semi[s])
            pltpu.async_copy(dst_h.at[pl.ds(off, _C)], di[s], semi[s])

        def wait_idx(s):
            pltpu.make_async_copy(src_h.at[pl.ds(0, _C)], si[s],
                                  semi[s]).wait()
            pltpu.make_async_copy(dst_h.at[pl.ds(0, _C)], di[s],
                                  semi[s]).wait()

        def issue_gathers(s):
            cps = []
            for t in range(ntab):
                cps.append(pltpu.async_copy(tabs[t].at[si[s]], bufs[s][2 * t],
                                            semg[s]))
                cps.append(pltpu.async_copy(tabs[t].at[di[s]],
                                            bufs[s][2 * t + 1], semg[s]))
            return cps

        def issue_wb(c, s):
            off = off_of(c)
            for t in range(l2):
                pltpu.async_copy(bufs[s][t], outs[t].at[pl.ds(off, _C)],
                                 semw[s])

        def drain_wb(s):
            for t in range(l2):
                pltpu.make_async_copy(bufs[s][t], outs[t].at[pl.ds(0, _C)],
                                      semw[s]).wait()

        issue_idx(0, 0)
        issue_idx(1, 1)

        def body(jj, c):
            a = 2 * jj
            b = a + 1

            @pl.when(jj > 0)
            def _():
                drain_wb(0)
                drain_wb(1)

            wait_idx(0)
            ga = issue_gathers(0)
            wait_idx(1)
            gb = issue_gathers(1)
            for d in ga:
                d.wait()
            issue_wb(a, 0)

            @pl.when(a + 2 < trips)
            def _():
                issue_idx(a + 2, 0)

            for d in gb:
                d.wait()
            issue_wb(b, 1)

            @pl.when(b + 2 < trips)
            def _():
                issue_idx(b + 2, 1)

            return c

        lax.fori_loop(0, pairs, body, 0)
        drain_wb(0)
        drain_wb(1)

        @pl.when(odd == 1)
        def _():
            wait_idx(0)
            for d in issue_gathers(0):
                d.wait()
            issue_wb(2 * pairs, 0)
            drain_wb(0)

    return k(*tables, src, dst)


def _sc_scatter(p, dst, n):
    e, wp = p.shape
    nch = e // _C
    base_tr, extra = nch // _NW, nch % _NW
    rpt = n // 16
    zr = 128
    nz = rpt // zr
    mesh = plsc.VectorSubcoreMesh(core_axis_name="c", subcore_axis_name="s")

    @functools.partial(
        pl.kernel,
        out_type=(jax.ShapeDtypeStruct((n, wp), F32),
                  jax.ShapeDtypeStruct((n, wp), F32)),
        mesh=mesh,
        compiler_params=pltpu.CompilerParams(use_tc_tiling_on_sc=False),
        scratch_types=[pltpu.VMEM((_C,), jnp.int32),
                       pltpu.VMEM((_C,), jnp.int32),
                       pltpu.VMEM((_C, wp), F32),
                       pltpu.VMEM((_C, wp), F32),
                       pltpu.VMEM((zr, wp), F32),
                       pltpu.VMEM_SHARED((n, wp), F32),
                       pltpu.SemaphoreType.DMA,
                       pltpu.SemaphoreType.DMA])
    def k(p_h, dst_h, o0_h, o1_h, i0_v, i1_v, b0_v, b1_v, zb_v, acc_s,
          sl0, sl1):
        idx_v = [i0_v, i1_v]
        buf_v = [b0_v, b1_v]
        seml = [sl0, sl1]
        cid = lax.axis_index("c")
        sid = lax.axis_index("s")
        wid = sid * 2 + cid
        row0 = sid * rpt

        def zb_body(i, c):
            zb_v[i, :] = jnp.zeros((wp,), F32)
            return c

        lax.fori_loop(0, zr, zb_body, 0)

        def z_body(kk, c):
            pltpu.sync_copy(zb_v, acc_s.at[pl.ds(row0 + kk * zr, zr)])
            return c

        lax.fori_loop(0, nz, z_body, 0)
        plsc.subcore_barrier()

        trips = base_tr + (wid < extra).astype(jnp.int32)
        pairs = trips // 2
        odd = trips - 2 * pairs

        def issue_loads(c, s):
            off = (wid + _NW * c) * _C
            pltpu.async_copy(dst_h.at[pl.ds(off, _C)], idx_v[s], seml[s])
            pltpu.async_copy(p_h.at[pl.ds(off, _C)], buf_v[s], seml[s])

        def wait_loads(s):
            pltpu.make_async_copy(dst_h.at[pl.ds(0, _C)], idx_v[s],
                                  seml[s]).wait()
            pltpu.make_async_copy(p_h.at[pl.ds(0, _C)], buf_v[s],
                                  seml[s]).wait()

        issue_loads(0, 0)
        issue_loads(1, 1)

        def body(jj, c):
            a = 2 * jj
            wait_loads(0)
            pltpu.sync_copy(buf_v[0], acc_s.at[idx_v[0]], add=True)

            @pl.when(a + 2 < trips)
            def _():
                issue_loads(a + 2, 0)

            wait_loads(1)
            pltpu.sync_copy(buf_v[1], acc_s.at[idx_v[1]], add=True)

            @pl.when(a + 3 < trips)
            def _():
                issue_loads(a + 3, 1)

            return c

        lax.fori_loop(0, pairs, body, 0)

        @pl.when(odd == 1)
        def _():
            wait_loads(0)
            pltpu.sync_copy(buf_v[0], acc_s.at[idx_v[0]], add=True)

        plsc.subcore_barrier()

        @pl.when(cid == 0)
        def _():
            pltpu.sync_copy(acc_s.at[pl.ds(row0, rpt)],
                            o0_h.at[pl.ds(row0, rpt)])

        @pl.when(cid == 1)
        def _():
            pltpu.sync_copy(acc_s.at[pl.ds(row0, rpt)],
                            o1_h.at[pl.ds(row0, rpt)])

    return k(p, dst)


def _kron8(w):
    return jnp.kron(jnp.eye(8, dtype=F32), w)


def _tile8(b):
    return jnp.tile(b.reshape(1, -1), (1, 8))


def _eye(r, c, off=0):
    return jnp.eye(r, c, off, dtype=F32)


def kernel(x, edge_attr, edge_index, params):
    n = x.shape[0]
    e = edge_attr.shape[0]
    npad = 51200
    n8 = npad // 8
    e8 = e // 8
    src = edge_index[0]
    dst = edge_index[1]
    be = 1000
    bnp = 800

    xp = jnp.pad(x, ((0, npad - n), (0, 0))).reshape(n8, 8 * x.shape[1])
    eap = edge_attr.reshape(e8, 8 * edge_attr.shape[1])

    x_carry = ("raw", xp)
    ea_carry = ("raw", eap)

    for i in range(5):
        ix, ox, ie, oe = _DIMS[2 * i]
        ixn, oxn, ien, oen = _DIMS[2 * i + 1]
        Wxe, bxe, Wee, bee = params[2 * i]
        Wxn, bxn, Wen, ben = params[2 * i + 1]
        ymode = ox <= ix
        w = ox if ymode else ix
        nhalf = 1 if w <= 16 else 2

        if x_carry[0] == "raw":
            row_ins = [x_carry[1]]
            pre = []
            n_parts = 1
            ixn_prev = x.shape[1]
            splits = [(0, ixn_prev)]
        else:
            _, x2p, p0, p1, ben_prev, oen_prev = x_carry
            oxn_prev = x2p.shape[1] // 8
            row_ins = [x2p, p0, p1]
            pre = [_kron8(_eye(_WSP, oen_prev)), _tile8(ben_prev)]
            n_parts = 2
            splits = [(0, oxn_prev), (oxn_prev, oxn_prev + oen_prev)]
        t_ws = []
        for t in range(nhalf):
            for (lo, hi) in splits:
                if ymode:
                    t_ws.append(_kron8(_pad_cols(
                        Wxe[lo:hi, 16 * t:16 * (t + 1)], 16)))
                else:
                    t_ws.append(_kron8(_eye(hi - lo, 16, lo - 16 * t)))
        n_ws = [_kron8(Wxn[lo:hi]) for (lo, hi) in splits]
        consts = pre + t_ws + n_ws + [_tile8(bxn)]

        def nk_fn(*vals, _np=n_parts, _nh=nhalf):
            vals = list(vals)
            if _np == 1:
                parts = [vals.pop(0)]
            else:
                x2p_b, p0_b, p1_b = vals[0], vals[1], vals[2]
                sel_b, bent_b = vals[3], vals[4]
                vals = vals[5:]
                xa = jnp.maximum(x2p_b, 0.0)
                xb = jnp.maximum(_dot(p0_b + p1_b, sel_b) + bent_b, 0.0)
                parts = [xa, xb]
            tw = [vals.pop(0) for _ in range(_nh * _np)]
            nw = [vals.pop(0) for _ in range(_np)]
            bxn_b = vals.pop(0)
            tables = [sum(_dot(p_, tw[t * _np + j])
                          for j, p_ in enumerate(parts))
                      for t in range(_nh)]
            x2 = sum(_dot(p_, w_) for p_, w_ in zip(parts, nw)) + bxn_b
            return tables + [x2]

        nk_outs = _rowmap(nk_fn, n8, bnp, row_ins, consts,
                          [128] * nhalf + [8 * oxn])
        tables = [tp.reshape(npad, 16) for tp in nk_outs[:nhalf]]
        x2 = nk_outs[nhalf]

        g_outs = _sc_gather(tables, src, dst)
        g_packed = [o.reshape(e8, 128) for o in g_outs]

        last = i == 4
        if ea_carry[0] == "raw":
            ea_ins = [ea_carry[1]]
            wee_parts = [_kron8(Wee)]
        else:
            _, a_prev, b_prev = ea_carry
            ea_ins = [a_prev, b_prev]
            sp = a_prev.shape[1] // 8
            wee_parts = [_kron8(Wee[:sp]), _kron8(Wee[sp:])]
        wx_half = []
        for t in range(nhalf):
            if ymode:
                wx_half.append(_kron8(_eye(16, ox, 16 * t)))
            else:
                blkw = Wxe[16 * t:16 * (t + 1)]
                wx_half.append(_kron8(jnp.pad(
                    blkw, ((0, 16 - blkw.shape[0]), (0, 0)))))
        e_consts = (wee_parts + [_tile8(bee)] + wx_half
                    + [_tile8(bxe),
                       _kron8(_pad_cols(Wen[:oe], _WSP)),
                       _kron8(_pad_cols(Wen[oe:], _WSP))])
        n_ea = len(ea_ins)

        def ek_fn(*vals, _n_ea=n_ea, _last=last, _nh=nhalf):
            vals = list(vals)
            ea_parts = [vals.pop(0) for _ in range(_n_ea)]
            g_bs = [vals.pop(0) for _ in range(2 * _nh)]
            wee_p = [vals.pop(0) for _ in range(_n_ea)]
            bee_b = vals.pop(0)
            wx_b = [vals.pop(0) for _ in range(_nh)]
            bxe_b = vals.pop(0)
            wna_b = vals.pop(0)
            wnb_b = vals.pop(0)
            ea2 = sum(_dot(ap, w_) for ap, w_ in zip(ea_parts, wee_p)) + bee_b
            ns = sum(_dot(g_bs[2 * t] + g_bs[2 * t + 1], wx_b[t])
                     for t in range(_nh)) + bxe_b
            a_ = jnp.maximum(ea2, 0.0)
            b_ = jnp.maximum(ns, 0.0)
            p_ = _dot(a_, wna_b) + _dot(b_, wnb_b)
            if _last:
                return [p_]
            return [a_, b_, p_]

        out_w = [8 * _WSP] if last else [8 * oe, 8 * ox, 8 * _WSP]
        ek_outs = _rowmap(ek_fn, e8, be, ea_ins + g_packed, e_consts, out_w)
        if last:
            p_pay = ek_outs[0]
            ea_carry = None
        else:
            a_new, b_new, p_pay = ek_outs
            ea_carry = ("parts", a_new, b_new)

        part0, part1 = _sc_scatter(p_pay.reshape(e, _WSP), dst, npad)
        p0 = part0.reshape(n8, 8 * _WSP)
        p1 = part1.reshape(n8, 8 * _WSP)
        x_carry = ("pend", x2, p0, p1, ben, oen)

    _, x2f, p0f, p1f, benf, oenf = x_carry
    oxnf = x2f.shape[1] // 8
    d5 = oxnf + oenf
    sel_es = _kron8(_eye(_WSP, oenf))
    ben_t = _tile8(benf)
    ka = _kron8(_eye(oxnf, d5))
    kb = _kron8(_eye(oenf, d5, oxnf))
    gsum = jnp.tile(_eye(d5, d5), (8, 1))
    kx = _kron8(_eye(d5, 2 * d5))
    ks = jnp.tile(_eye(d5, 2 * d5, d5), (1, 8))
    n8_valid = n // 8

    grid = (n8 // bnp,)

    def f1_body(x2_r, p0_r, p1_r, sel_r, bent_r, ka_r, kb_r, o5_r, s_r):
        xa = jnp.maximum(x2_r[...], 0.0)
        xb = jnp.maximum(_dot(p0_r[...] + p1_r[...], sel_r[...])
                         + bent_r[...], 0.0)
        x5 = _dot(xa, ka_r[...]) + _dot(xb, kb_r[...])
        o5_r[...] = x5
        rid = (pl.program_id(0) * bnp
               + jax.lax.broadcasted_iota(jnp.int32, (bnp, 1), 0))
        x5m = jnp.where(rid < n8_valid, x5, 0.0)

        @pl.when(pl.program_id(0) == 0)
        def _():
            s_r[...] = jnp.zeros_like(s_r)

        s_r[...] += jnp.sum(x5m, axis=0, keepdims=True)

    o5, ssum = pl.pallas_call(
        f1_body, grid=grid,
        in_specs=[pl.BlockSpec((bnp, x2f.shape[1]), lambda g: (g, 0)),
                  pl.BlockSpec((bnp, 8 * _WSP), lambda g: (g, 0)),
                  pl.BlockSpec((bnp, 8 * _WSP), lambda g: (g, 0)),
                  pl.BlockSpec(sel_es.shape, lambda g: (0, 0)),
                  pl.BlockSpec(ben_t.shape, lambda g: (0, 0)),
                  pl.BlockSpec(ka.shape, lambda g: (0, 0)),
                  pl.BlockSpec(kb.shape, lambda g: (0, 0))],
        out_specs=[pl.BlockSpec((bnp, 8 * d5), lambda g: (g, 0)),
                   pl.BlockSpec((1, 8 * d5), lambda g: (0, 0))],
        out_shape=[jax.ShapeDtypeStruct((n8, 8 * d5), F32),
                   jax.ShapeDtypeStruct((1, 8 * d5), F32)],
    )(x2f, p0f, p1f, sel_es, ben_t, ka, kb)

    def f2_body(o5_r, s_r, gs_r, kx_r, ks_r, out_r):
        s5 = _dot(s_r[...], gs_r[...])
        out_r[...] = _dot(o5_r[...], kx_r[...]) + jnp.broadcast_to(
            _dot(s5, ks_r[...]), (bnp, 16 * d5))

    outp = pl.pallas_call(
        f2_body, grid=grid,
        in_specs=[pl.BlockSpec((bnp, 8 * d5), lambda g: (g, 0)),
                  pl.BlockSpec((1, 8 * d5), lambda g: (0, 0)),
                  pl.BlockSpec(gsum.shape, lambda g: (0, 0)),
                  pl.BlockSpec(kx.shape, lambda g: (0, 0)),
                  pl.BlockSpec(ks.shape, lambda g: (0, 0))],
        out_specs=pl.BlockSpec((bnp, 16 * d5), lambda g: (g, 0)),
        out_shape=jax.ShapeDtypeStruct((n8, 16 * d5), F32),
    )(o5, ssum, gsum, kx, ks)
    return outp.reshape(npad, 2 * d5)[:n]


# speedup vs baseline: 8.7926x; 1.7742x over previous
import functools

import jax
import jax.numpy as jnp
from jax import lax
from jax.experimental import pallas as pl
from jax.experimental.pallas import tpu as pltpu
from jax.experimental.pallas import tpu_sc as plsc

F32 = jnp.float32
BF16 = jnp.bfloat16
_HI = jax.lax.Precision.HIGHEST

_DIMS = [(2, 2, 1, 2), (2, 2, 4, 5), (7, 9, 4, 5), (7, 9, 14, 15),
         (24, 30, 14, 15), (24, 30, 45, 15), (45, 20, 45, 15),
         (45, 20, 35, 10), (30, 3, 35, 2), (30, 3, 5, 2)]

_C = 128
_NW = 32
_WSP = 16


def _dot(a, b):
    return jax.lax.dot_general(a, b, (((1,), (0,)), ((), ())),
                               precision=_HI, preferred_element_type=F32)


def _d1(a, w_bf):
    return jax.lax.dot_general(a.astype(BF16), w_bf,
                               (((1,), (0,)), ((), ())),
                               precision=jax.lax.Precision.DEFAULT,
                               preferred_element_type=F32)


def _pad_cols(w, wp):
    return jnp.pad(w, ((0, 0), (0, wp - w.shape[1])))


def _p16r(m):
    return jnp.pad(m, ((0, 16 - m.shape[0]), (0, 0)))


def _kron8(w):
    return jnp.kron(jnp.eye(8, dtype=F32), w)


def _tile8(b):
    return jnp.tile(b.reshape(1, -1), (1, 8))


def _eye(r, c, off=0):
    return jnp.eye(r, c, off, dtype=F32)


def _rowmap(fn, nrows, blk, row_ins, consts, out_widths):
    grid = (nrows // blk,)
    in_specs = (
        [pl.BlockSpec((blk, a.shape[1]), lambda g: (g, 0)) for a in row_ins]
        + [pl.BlockSpec(c.shape, lambda g: (0, 0)) for c in consts]
    )
    out_specs = [pl.BlockSpec((blk, w), lambda g: (g, 0)) for w in out_widths]
    out_shape = [jax.ShapeDtypeStruct((nrows, w), F32) for w in out_widths]
    nin = len(row_ins) + len(consts)

    def body(*refs):
        vals = [r[...] for r in refs[:nin]]
        outs = fn(*vals)
        for oref, o in zip(refs[nin:], outs):
            oref[...] = o

    return pl.pallas_call(body, grid=grid, in_specs=in_specs,
                          out_specs=out_specs, out_shape=out_shape)(
        *row_ins, *consts)


def _sc_gather(tables, src, dst):
    ntab = len(tables)
    e = src.shape[0]
    nch = e // _C
    base_tr, extra = nch // _NW, nch % _NW
    mesh = plsc.VectorSubcoreMesh(core_axis_name="c", subcore_axis_name="s")
    out_t = tuple(jax.ShapeDtypeStruct((e, 16), F32) for _ in range(2 * ntab))
    l2 = 2 * ntab

    @functools.partial(
        pl.kernel, out_type=out_t, mesh=mesh,
        compiler_params=pltpu.CompilerParams(use_tc_tiling_on_sc=False),
        scratch_types=([pltpu.VMEM((_C,), jnp.int32)] * 4
                       + [pltpu.VMEM((_C, 16), F32)] * (2 * l2)
                       + [pltpu.SemaphoreType.DMA] * 6))
    def k(*refs):
        tabs = refs[:ntab]
        src_h, dst_h = refs[ntab], refs[ntab + 1]
        outs = refs[ntab + 2:ntab + 2 + l2]
        sc = list(refs[ntab + 2 + l2:])
        si = [sc[0], sc[2]]
        di = [sc[1], sc[3]]
        bufs = [sc[4:4 + l2], sc[4 + l2:4 + 2 * l2]]
        semi = [sc[4 + 2 * l2], sc[5 + 2 * l2]]
        semg = [sc[6 + 2 * l2], sc[7 + 2 * l2]]
        semw = [sc[8 + 2 * l2], sc[9 + 2 * l2]]
        cid = lax.axis_index("c")
        sid = lax.axis_index("s")
        wid = sid * 2 + cid
        trips = base_tr + (wid < extra).astype(jnp.int32)
        pairs = trips // 2
        odd = trips - 2 * pairs

        def off_of(c):
            return (wid + _NW * c) * _C

        def issue_idx(c, s):
            off = off_of(c)
            pltpu.async_copy(src_h.at[pl.ds(off, _C)], si[s], semi[s])
            pltpu.async_copy(dst_h.at[pl.ds(off, _C)], di[s], semi[s])

        def wait_idx(s):
            pltpu.make_async_copy(src_h.at[pl.ds(0, _C)], si[s],
                                  semi[s]).wait()
            pltpu.make_async_copy(dst_h.at[pl.ds(0, _C)], di[s],
                                  semi[s]).wait()

        def issue_gathers(s):
            cps = []
            for t in range(ntab):
                cps.append(pltpu.async_copy(tabs[t].at[si[s]], bufs[s][2 * t],
                                            semg[s]))
                cps.append(pltpu.async_copy(tabs[t].at[di[s]],
                                            bufs[s][2 * t + 1], semg[s]))
            return cps

        def issue_wb(c, s):
            off = off_of(c)
            for t in range(l2):
                pltpu.async_copy(bufs[s][t], outs[t].at[pl.ds(off, _C)],
                                 semw[s])

        def drain_wb(s):
            for t in range(l2):
                pltpu.make_async_copy(bufs[s][t], outs[t].at[pl.ds(0, _C)],
                                      semw[s]).wait()

        issue_idx(0, 0)
        issue_idx(1, 1)

        def body(jj, c):
            a = 2 * jj
            b = a + 1

            @pl.when(jj > 0)
            def _():
                drain_wb(0)
                drain_wb(1)

            wait_idx(0)
            ga = issue_gathers(0)
            wait_idx(1)
            gb = issue_gathers(1)
            for d in ga:
                d.wait()
            issue_wb(a, 0)

            @pl.when(a + 2 < trips)
            def _():
                issue_idx(a + 2, 0)

            for d in gb:
                d.wait()
            issue_wb(b, 1)

            @pl.when(b + 2 < trips)
            def _():
                issue_idx(b + 2, 1)

            return c

        lax.fori_loop(0, pairs, body, 0)
        drain_wb(0)
        drain_wb(1)

        @pl.when(odd == 1)
        def _():
            wait_idx(0)
            for d in issue_gathers(0):
                d.wait()
            issue_wb(2 * pairs, 0)
            drain_wb(0)

    return k(*tables, src, dst)


def _sc_scatter(pays, dst, n):
    npay = len(pays)
    e = dst.shape[0]
    nch = e // _C
    base_tr, extra = nch // _NW, nch % _NW
    rpt = n // 16
    zr = 128
    nz = rpt // zr
    mesh = plsc.VectorSubcoreMesh(core_axis_name="c", subcore_axis_name="s")
    out_t = tuple(jax.ShapeDtypeStruct((n, 16), F32)
                  for _ in range(2 * npay))

    @functools.partial(
        pl.kernel, out_type=out_t, mesh=mesh,
        compiler_params=pltpu.CompilerParams(use_tc_tiling_on_sc=False),
        scratch_types=([pltpu.VMEM((_C,), jnp.int32)] * 2
                       + [pltpu.VMEM((_C, 16), F32)] * (2 * npay)
                       + [pltpu.VMEM((zr, 16), F32)]
                       + [pltpu.VMEM_SHARED((n, 16), F32)] * npay
                       + [pltpu.SemaphoreType.DMA] * 2))
    def k(*refs):
        pay_h = refs[:npay]
        dst_h = refs[npay]
        outs = refs[npay + 1:npay + 1 + 2 * npay]
        sc = list(refs[npay + 1 + 2 * npay:])
        idx_v = [sc[0], sc[1]]
        buf_v = [sc[2:2 + npay], sc[2 + npay:2 + 2 * npay]]
        zb_v = sc[2 + 2 * npay]
        accs = sc[3 + 2 * npay:3 + 3 * npay]
        seml = sc[3 + 3 * npay:]
        cid = lax.axis_index("c")
        sid = lax.axis_index("s")
        wid = sid * 2 + cid
        row0 = sid * rpt

        def zb_body(i, c):
            zb_v[i, :] = jnp.zeros((16,), F32)
            return c

        lax.fori_loop(0, zr, zb_body, 0)

        def z_body(kk, c):
            for acc in accs:
                pltpu.sync_copy(zb_v, acc.at[pl.ds(row0 + kk * zr, zr)])
            return c

        lax.fori_loop(0, nz, z_body, 0)
        plsc.subcore_barrier()

        trips = base_tr + (wid < extra).astype(jnp.int32)
        pairs = trips // 2
        odd = trips - 2 * pairs

        def issue_loads(c, s):
            off = (wid + _NW * c) * _C
            pltpu.async_copy(dst_h.at[pl.ds(off, _C)], idx_v[s], seml[s])
            for q in range(npay):
                pltpu.async_copy(pay_h[q].at[pl.ds(off, _C)], buf_v[s][q],
                                 seml[s])

        def wait_loads(s):
            pltpu.make_async_copy(dst_h.at[pl.ds(0, _C)], idx_v[s],
                                  seml[s]).wait()
            for q in range(npay):
                pltpu.make_async_copy(pay_h[q].at[pl.ds(0, _C)], buf_v[s][q],
                                      seml[s]).wait()

        def do_scatter(s):
            for q in range(npay):
                pltpu.sync_copy(buf_v[s][q], accs[q].at[idx_v[s]], add=True)

        issue_loads(0, 0)
        issue_loads(1, 1)

        def body(jj, c):
            a = 2 * jj
            wait_loads(0)
            do_scatter(0)

            @pl.when(a + 2 < trips)
            def _():
                issue_loads(a + 2, 0)

            wait_loads(1)
            do_scatter(1)

            @pl.when(a + 3 < trips)
            def _():
                issue_loads(a + 3, 1)

            return c

        lax.fori_loop(0, pairs, body, 0)

        @pl.when(odd == 1)
        def _():
            wait_loads(0)
            do_scatter(0)

        plsc.subcore_barrier()

        for q in range(npay):
            @pl.when(cid == 0)
            def _(q=q):
                pltpu.sync_copy(accs[q].at[pl.ds(row0, rpt)],
                                outs[2 * q].at[pl.ds(row0, rpt)])

            @pl.when(cid == 1)
            def _(q=q):
                pltpu.sync_copy(accs[q].at[pl.ds(row0, rpt)],
                                outs[2 * q + 1].at[pl.ds(row0, rpt)])

    return k(*pays, dst)


def kernel(x, edge_attr, edge_index, params):
    n = x.shape[0]
    e = edge_attr.shape[0]
    npad = 51200
    n8 = npad // 8
    e8 = e // 8
    src = edge_index[0]
    dst = edge_index[1]
    be = 1000
    bnp = 800
    bf = lambda m: m.astype(BF16)

    xp = jnp.pad(x, ((0, npad - n), (0, 0))).reshape(n8, 8 * x.shape[1])
    eap = edge_attr.reshape(e8, 8 * edge_attr.shape[1])

    x_carry = ("raw", xp)
    ea_halves = None

    for i in range(5):
        ix, ox, ie, oe = _DIMS[2 * i]
        ixn, oxn, ien, oen = _DIMS[2 * i + 1]
        Wxe, bxe, Wee, bee = params[2 * i]
        Wxn, bxn, Wen, ben = params[2 * i + 1]
        nhg = (ix + 15) // 16
        nbh = (ox + 15) // 16
        nsc = 1 + nbh

        if x_carry[0] == "raw":
            row_ins = [x_carry[1]]
            consts = []
            n_parts = 1
            plens = [ix]
            poffs = [0]
            nscp = 0
        else:
            (_, x2p, parts_sc, Wen_p, ben_p, oe_p, ie_p, oxn_p,
             oen_p) = x_carry
            nscp = len(parts_sc) // 2
            row_ins = [x2p] + parts_sc
            wen_halves = [bf(_kron8(_p16r(Wen_p[:oe_p])))]
            for t in range(nscp - 1):
                wen_halves.append(bf(_kron8(_p16r(
                    Wen_p[oe_p + 16 * t:min(oe_p + 16 * (t + 1), ie_p)]))))
            consts = wen_halves + [_tile8(ben_p)]
            n_parts = 2
            plens = [oxn_p, oen_p]
            poffs = [0, oxn_p]
        t_ws = []
        for t in range(nhg):
            for j in range(n_parts):
                t_ws.append(_kron8(_eye(plens[j], 16, poffs[j] - 16 * t)))
        n_ws = [bf(_kron8(Wxn[po:po + pl_])) for po, pl_ in zip(poffs, plens)]
        consts = consts + t_ws + n_ws + [_tile8(bxn)]

        def nk_fn(*vals, _np=n_parts, _nh=nhg, _nscp=nscp):
            vals = list(vals)
            if _np == 1:
                parts = [vals.pop(0)]
            else:
                x2p_b = vals.pop(0)
                es_parts = [vals.pop(0) for _ in range(2 * _nscp)]
                wenh = [vals.pop(0) for _ in range(_nscp)]
                benp_b = vals.pop(0)
                xa = jnp.maximum(x2p_b, 0.0)
                es_m = sum(_d1(es_parts[2 * h] + es_parts[2 * h + 1],
                               wenh[h]) for h in range(_nscp))
                xb = jnp.maximum(es_m + benp_b, 0.0)
                parts = [xa, xb]
            tw = [vals.pop(0) for _ in range(_nh * _np)]
            nw = [vals.pop(0) for _ in range(_np)]
            bxn_b = vals.pop(0)
            tables = [sum(_dot(p_, tw[t * _np + j])
                          for j, p_ in enumerate(parts))
                      for t in range(_nh)]
            x2 = sum(_d1(p_, w_) for p_, w_ in zip(parts, nw)) + bxn_b
            return tables + [x2]

        nk_outs = _rowmap(nk_fn, n8, bnp, row_ins, consts,
                          [128] * nhg + [8 * oxn])
        tables = [tp.reshape(npad, 16) for tp in nk_outs[:nhg]]
        x2 = nk_outs[nhg]

        g_outs = _sc_gather(tables, src, dst)
        g_packed = [o.reshape(e8, 128) for o in g_outs]

        if ea_halves is None:
            ea_ins = [eap]
            wee_parts = [bf(_kron8(_pad_cols(Wee, 16)))]
            oe_prev = None
        else:
            ea_ins = ea_halves
            wee_parts = [bf(_kron8(_pad_cols(_p16r(Wee[:oe_prev]), 16)))]
            for t in range(len(ea_halves) - 1):
                rows = Wee[oe_prev + 16 * t:min(oe_prev + 16 * (t + 1), ie)]
                wee_parts.append(bf(_kron8(_pad_cols(_p16r(rows), 16))))
        bee_t = _tile8(jnp.pad(bee.reshape(-1), (0, 16 - oe)))
        bxe_halves = [
            _tile8(jnp.pad(bxe[16 * t:16 * (t + 1)],
                           (0, 16 * (t + 1) - min(16 * (t + 1), ox))))
            for t in range(nbh)]
        wx_st = []
        for s in range(nhg):
            for t in range(nbh):
                blkw = Wxe[16 * s:16 * (s + 1), 16 * t:16 * (t + 1)]
                wx_st.append(bf(_kron8(_pad_cols(_p16r(blkw), 16))))
        e_consts = wee_parts + [bee_t] + wx_st + bxe_halves
        n_ea = len(ea_ins)

        def ek_fn(*vals, _n_ea=n_ea, _nh=nhg, _nb=nbh):
            vals = list(vals)
            ea_parts = [vals.pop(0) for _ in range(_n_ea)]
            g_bs = [vals.pop(0) for _ in range(2 * _nh)]
            wee_p = [vals.pop(0) for _ in range(_n_ea)]
            bee_b = vals.pop(0)
            wx_b = [vals.pop(0) for _ in range(_nh * _nb)]
            bxe_b = [vals.pop(0) for _ in range(_nb)]
            ea2 = sum(_d1(ap, w_) for ap, w_ in zip(ea_parts, wee_p)) + bee_b
            a_ = jnp.maximum(ea2, 0.0)
            gsum = [g_bs[2 * t] + g_bs[2 * t + 1] for t in range(_nh)]
            outs = [a_]
            for t in range(_nb):
                nst = sum(_d1(gsum[s], wx_b[s * _nb + t])
                          for s in range(_nh)) + bxe_b[t]
                outs.append(jnp.maximum(nst, 0.0))
            return outs

        ek_outs = _rowmap(ek_fn, e8, be, ea_ins + g_packed, e_consts,
                          [128] * nsc)
        ea_halves = list(ek_outs)
        oe_prev = oe

        pays = [h.reshape(e, _WSP) for h in ea_halves]
        parts_sc = []
        q = 0
        while q < nsc:
            grp = pays[q:q + 2]
            outs = _sc_scatter(grp, dst, npad)
            parts_sc.extend(o.reshape(n8, 128) for o in outs)
            q += 2
        x_carry = ("pend", x2, parts_sc, Wen, ben, oe, ien, oxn, oen)

    (_, x2f, parts_f, Wen_f, ben_f, oe_f, ie_f, oxnf, oenf) = x_carry
    nscf = len(parts_f) // 2
    d5 = oxnf + oenf
    wenf_halves = [bf(_kron8(_p16r(Wen_f[:oe_f])))]
    for t in range(nscf - 1):
        wenf_halves.append(bf(_kron8(_p16r(
            Wen_f[oe_f + 16 * t:min(oe_f + 16 * (t + 1), ie_f)]))))
    benf_t = _tile8(ben_f)
    ka = _kron8(_eye(oxnf, d5))
    kb = _kron8(_eye(oenf, d5, oxnf))
    gsum_m = jnp.tile(_eye(d5, d5), (8, 1))
    kx = _kron8(_eye(d5, 2 * d5))
    ks = jnp.tile(_eye(d5, 2 * d5, d5), (1, 8))
    n8_valid = n // 8
    grid = (n8 // bnp,)

    def f1_body(*refs):
        x2_r = refs[0]
        es_rs = refs[1:1 + 2 * nscf]
        wenh = refs[1 + 2 * nscf:1 + 3 * nscf]
        bent_r, ka_r, kb_r = refs[1 + 3 * nscf:4 + 3 * nscf]
        o5_r, s_r = refs[4 + 3 * nscf:]
        xa = jnp.maximum(x2_r[...], 0.0)
        es_m = sum(_d1(es_rs[2 * h][...] + es_rs[2 * h + 1][...],
                       wenh[h][...]) for h in range(nscf))
        xb = jnp.maximum(es_m + bent_r[...], 0.0)
        x5 = _dot(xa, ka_r[...]) + _dot(xb, kb_r[...])
        o5_r[...] = x5
        rid = (pl.program_id(0) * bnp
               + jax.lax.broadcasted_iota(jnp.int32, (bnp, 1), 0))
        x5m = jnp.where(rid < n8_valid, x5, 0.0)

        @pl.when(pl.program_id(0) == 0)
        def _():
            s_r[...] = jnp.zeros_like(s_r)

        s_r[...] += jnp.sum(x5m, axis=0, keepdims=True)

    f1_consts = wenf_halves + [benf_t, ka, kb]
    o5, ssum = pl.pallas_call(
        f1_body, grid=grid,
        in_specs=([pl.BlockSpec((bnp, x2f.shape[1]), lambda g: (g, 0))]
                  + [pl.BlockSpec((bnp, 128), lambda g: (g, 0))
                     for _ in parts_f]
                  + [pl.BlockSpec(c.shape, lambda g: (0, 0))
                     for c in f1_consts]),
        out_specs=[pl.BlockSpec((bnp, 8 * d5), lambda g: (g, 0)),
                   pl.BlockSpec((1, 8 * d5), lambda g: (0, 0))],
        out_shape=[jax.ShapeDtypeStruct((n8, 8 * d5), F32),
                   jax.ShapeDtypeStruct((1, 8 * d5), F32)],
    )(x2f, *parts_f, *f1_consts)

    def f2_body(o5_r, s_r, gs_r, kx_r, ks_r, out_r):
        s5 = _dot(s_r[...], gs_r[...])
        out_r[...] = _dot(o5_r[...], kx_r[...]) + jnp.broadcast_to(
            _dot(s5, ks_r[...]), (bnp, 16 * d5))

    outp = pl.pallas_call(
        f2_body, grid=grid,
        in_specs=[pl.BlockSpec((bnp, 8 * d5), lambda g: (g, 0)),
                  pl.BlockSpec((1, 8 * d5), lambda g: (0, 0)),
                  pl.BlockSpec(gsum_m.shape, lambda g: (0, 0)),
                  pl.BlockSpec(kx.shape, lambda g: (0, 0)),
                  pl.BlockSpec(ks.shape, lambda g: (0, 0))],
        out_specs=pl.BlockSpec((bnp, 16 * d5), lambda g: (g, 0)),
        out_shape=jax.ShapeDtypeStruct((n8, 16 * d5), F32),
    )(o5, ssum, gsum_m, kx, ks)
    return outp.reshape(npad, 2 * d5)[:n]


# speedup vs baseline: 9.2135x; 1.0479x over previous
import functools

import jax
import jax.numpy as jnp
from jax import lax
from jax.experimental import pallas as pl
from jax.experimental.pallas import tpu as pltpu
from jax.experimental.pallas import tpu_sc as plsc

F32 = jnp.float32
BF16 = jnp.bfloat16
_HI = jax.lax.Precision.HIGHEST

_DIMS = [(2, 2, 1, 2), (2, 2, 4, 5), (7, 9, 4, 5), (7, 9, 14, 15),
         (24, 30, 14, 15), (24, 30, 45, 15), (45, 20, 45, 15),
         (45, 20, 35, 10), (30, 3, 35, 2), (30, 3, 5, 2)]

_C = 128
_NW = 32
_WSP = 16


def _dot(a, b):
    return jax.lax.dot_general(a, b, (((1,), (0,)), ((), ())),
                               precision=_HI, preferred_element_type=F32)


def _d1(a, w_bf):
    return jax.lax.dot_general(a.astype(BF16), w_bf,
                               (((1,), (0,)), ((), ())),
                               precision=jax.lax.Precision.DEFAULT,
                               preferred_element_type=F32)


def _pad_cols(w, wp):
    return jnp.pad(w, ((0, 0), (0, wp - w.shape[1])))


def _p16r(m):
    return jnp.pad(m, ((0, 16 - m.shape[0]), (0, 0)))


def _kron8(w):
    return jnp.kron(jnp.eye(8, dtype=F32), w)


def _tile8(b):
    return jnp.tile(b.reshape(1, -1), (1, 8))


def _eye(r, c, off=0):
    return jnp.eye(r, c, off, dtype=F32)


def _rowmap(fn, nrows, blk, row_ins, consts, out_widths):
    grid = (nrows // blk,)
    in_specs = (
        [pl.BlockSpec((blk, a.shape[1]), lambda g: (g, 0)) for a in row_ins]
        + [pl.BlockSpec(c.shape, lambda g: (0, 0)) for c in consts]
    )
    out_specs = [pl.BlockSpec((blk, w), lambda g: (g, 0)) for w in out_widths]
    out_shape = [jax.ShapeDtypeStruct((nrows, w), F32) for w in out_widths]
    nin = len(row_ins) + len(consts)

    def body(*refs):
        vals = [r[...] for r in refs[:nin]]
        outs = fn(*vals)
        for oref, o in zip(refs[nin:], outs):
            oref[...] = o

    return pl.pallas_call(body, grid=grid, in_specs=in_specs,
                          out_specs=out_specs, out_shape=out_shape)(
        *row_ins, *consts)


def _sc_gather(tables, src, dst):
    ntab = len(tables)
    e = src.shape[0]
    nch = e // _C
    base_tr, extra = nch // _NW, nch % _NW
    mesh = plsc.VectorSubcoreMesh(core_axis_name="c", subcore_axis_name="s")
    out_t = tuple(jax.ShapeDtypeStruct((e, 16), F32) for _ in range(2 * ntab))
    l2 = 2 * ntab

    @functools.partial(
        pl.kernel, out_type=out_t, mesh=mesh,
        compiler_params=pltpu.CompilerParams(use_tc_tiling_on_sc=False),
        scratch_types=([pltpu.VMEM((_C,), jnp.int32)] * 4
                       + [pltpu.VMEM((_C, 16), F32)] * (2 * l2)
                       + [pltpu.SemaphoreType.DMA] * 6))
    def k(*refs):
        tabs = refs[:ntab]
        src_h, dst_h = refs[ntab], refs[ntab + 1]
        outs = refs[ntab + 2:ntab + 2 + l2]
        sc = list(refs[ntab + 2 + l2:])
        si = [sc[0], sc[2]]
        di = [sc[1], sc[3]]
        bufs = [sc[4:4 + l2], sc[4 + l2:4 + 2 * l2]]
        semi = [sc[4 + 2 * l2], sc[5 + 2 * l2]]
        semg = [sc[6 + 2 * l2], sc[7 + 2 * l2]]
        semw = [sc[8 + 2 * l2], sc[9 + 2 * l2]]
        cid = lax.axis_index("c")
        sid = lax.axis_index("s")
        wid = sid * 2 + cid
        trips = base_tr + (wid < extra).astype(jnp.int32)
        pairs = trips // 2
        odd = trips - 2 * pairs

        def off_of(c):
            return (wid + _NW * c) * _C

        def issue_idx(c, s):
            off = off_of(c)
            pltpu.async_copy(src_h.at[pl.ds(off, _C)], si[s], semi[s])
            pltpu.async_copy(dst_h.at[pl.ds(off, _C)], di[s], semi[s])

        def wait_idx(s):
            pltpu.make_async_copy(src_h.at[pl.ds(0, _C)], si[s],
                                  semi[s]).wait()
            pltpu.make_async_copy(dst_h.at[pl.ds(0, _C)], di[s],
                                  semi[s]).wait()

        def issue_gathers(s):
            cps = []
            for t in range(ntab):
                cps.append(pltpu.async_copy(tabs[t].at[si[s]], bufs[s][2 * t],
                                            semg[s]))
                cps.append(pltpu.async_copy(tabs[t].at[di[s]],
                                            bufs[s][2 * t + 1], semg[s]))
            return cps

        def issue_wb(c, s):
            off = off_of(c)
            for t in range(l2):
                pltpu.async_copy(bufs[s][t], outs[t].at[pl.ds(off, _C)],
                                 semw[s])

        def drain_wb(s):
            for t in range(l2):
                pltpu.make_async_copy(bufs[s][t], outs[t].at[pl.ds(0, _C)],
                                      semw[s]).wait()

        issue_idx(0, 0)
        issue_idx(1, 1)

        def body(jj, c):
            a = 2 * jj
            b = a + 1

            @pl.when(jj > 0)
            def _():
                drain_wb(0)
                drain_wb(1)

            wait_idx(0)
            ga = issue_gathers(0)
            wait_idx(1)
            gb = issue_gathers(1)
            for d in ga:
                d.wait()
            issue_wb(a, 0)

            @pl.when(a + 2 < trips)
            def _():
                issue_idx(a + 2, 0)

            for d in gb:
                d.wait()
            issue_wb(b, 1)

            @pl.when(b + 2 < trips)
            def _():
                issue_idx(b + 2, 1)

            return c

        lax.fori_loop(0, pairs, body, 0)
        drain_wb(0)
        drain_wb(1)

        @pl.when(odd == 1)
        def _():
            wait_idx(0)
            for d in issue_gathers(0):
                d.wait()
            issue_wb(2 * pairs, 0)
            drain_wb(0)

    return k(*tables, src, dst)


def _sc_scatter(pays, dst, n):
    npay = len(pays)
    e = dst.shape[0]
    nch = e // _C
    base_tr, extra = nch // _NW, nch % _NW
    rpt = n // 16
    zr = 128
    nz = rpt // zr
    mesh = plsc.VectorSubcoreMesh(core_axis_name="c", subcore_axis_name="s")
    out_t = tuple(jax.ShapeDtypeStruct((n, 16), F32)
                  for _ in range(2 * npay))

    @functools.partial(
        pl.kernel, out_type=out_t, mesh=mesh,
        compiler_params=pltpu.CompilerParams(use_tc_tiling_on_sc=False),
        scratch_types=([pltpu.VMEM((_C,), jnp.int32)] * 2
                       + [pltpu.VMEM((_C, 16), F32)] * (2 * npay)
                       + [pltpu.VMEM((zr, 16), F32)]
                       + [pltpu.VMEM_SHARED((n, 16), F32)] * npay
                       + [pltpu.SemaphoreType.DMA] * 2))
    def k(*refs):
        pay_h = refs[:npay]
        dst_h = refs[npay]
        outs = refs[npay + 1:npay + 1 + 2 * npay]
        sc = list(refs[npay + 1 + 2 * npay:])
        idx_v = [sc[0], sc[1]]
        buf_v = [sc[2:2 + npay], sc[2 + npay:2 + 2 * npay]]
        zb_v = sc[2 + 2 * npay]
        accs = sc[3 + 2 * npay:3 + 3 * npay]
        seml = sc[3 + 3 * npay:]
        cid = lax.axis_index("c")
        sid = lax.axis_index("s")
        wid = sid * 2 + cid
        row0 = sid * rpt

        def zb_body(i, c):
            zb_v[i, :] = jnp.zeros((16,), F32)
            return c

        lax.fori_loop(0, zr, zb_body, 0)

        def z_body(kk, c):
            for acc in accs:
                pltpu.sync_copy(zb_v, acc.at[pl.ds(row0 + kk * zr, zr)])
            return c

        lax.fori_loop(0, nz, z_body, 0)
        plsc.subcore_barrier()

        trips = base_tr + (wid < extra).astype(jnp.int32)
        pairs = trips // 2
        odd = trips - 2 * pairs

        def issue_loads(c, s):
            off = (wid + _NW * c) * _C
            pltpu.async_copy(dst_h.at[pl.ds(off, _C)], idx_v[s], seml[s])
            for q in range(npay):
                pltpu.async_copy(pay_h[q].at[pl.ds(off, _C)], buf_v[s][q],
                                 seml[s])

        def wait_loads(s):
            pltpu.make_async_copy(dst_h.at[pl.ds(0, _C)], idx_v[s],
                                  seml[s]).wait()
            for q in range(npay):
                pltpu.make_async_copy(pay_h[q].at[pl.ds(0, _C)], buf_v[s][q],
                                      seml[s]).wait()

        def do_scatter(s):
            for q in range(npay):
                pltpu.sync_copy(buf_v[s][q], accs[q].at[idx_v[s]], add=True)

        issue_loads(0, 0)
        issue_loads(1, 1)

        def body(jj, c):
            a = 2 * jj
            wait_loads(0)
            do_scatter(0)

            @pl.when(a + 2 < trips)
            def _():
                issue_loads(a + 2, 0)

            wait_loads(1)
            do_scatter(1)

            @pl.when(a + 3 < trips)
            def _():
                issue_loads(a + 3, 1)

            return c

        lax.fori_loop(0, pairs, body, 0)

        @pl.when(odd == 1)
        def _():
            wait_loads(0)
            do_scatter(0)

        plsc.subcore_barrier()

        for q in range(npay):
            @pl.when(cid == 0)
            def _(q=q):
                pltpu.sync_copy(accs[q].at[pl.ds(row0, rpt)],
                                outs[2 * q].at[pl.ds(row0, rpt)])

            @pl.when(cid == 1)
            def _(q=q):
                pltpu.sync_copy(accs[q].at[pl.ds(row0, rpt)],
                                outs[2 * q + 1].at[pl.ds(row0, rpt)])

    return k(*pays, dst)


def kernel(x, edge_attr, edge_index, params):
    n = x.shape[0]
    e = edge_attr.shape[0]
    npad = 51200
    n8 = npad // 8
    e8 = e // 8
    src = edge_index[0]
    dst = edge_index[1]
    be = 2000
    bnp = 800
    bf = lambda m: m.astype(BF16)

    xp = jnp.pad(x, ((0, npad - n), (0, 0))).reshape(n8, 8 * x.shape[1])
    eap = edge_attr.reshape(e8, 8 * edge_attr.shape[1])

    x_carry = ("raw", xp)
    ea_halves = None

    for i in range(5):
        ix, ox, ie, oe = _DIMS[2 * i]
        ixn, oxn, ien, oen = _DIMS[2 * i + 1]
        Wxe, bxe, Wee, bee = params[2 * i]
        Wxn, bxn, Wen, ben = params[2 * i + 1]
        nhg = (ix + 15) // 16
        nbh = (ox + 15) // 16
        nsc = 1 + nbh

        if x_carry[0] == "raw":
            row_ins = [x_carry[1]]
            consts = []
            n_parts = 1
            plens = [ix]
            poffs = [0]
            nscp = 0
        else:
            (_, x2p, parts_sc, Wen_p, ben_p, oe_p, ie_p, oxn_p,
             oen_p) = x_carry
            nscp = len(parts_sc) // 2
            row_ins = [x2p] + parts_sc
            wen_halves = [bf(_kron8(_p16r(Wen_p[:oe_p])))]
            for t in range(nscp - 1):
                wen_halves.append(bf(_kron8(_p16r(
                    Wen_p[oe_p + 16 * t:min(oe_p + 16 * (t + 1), ie_p)]))))
            consts = wen_halves + [_tile8(ben_p)]
            n_parts = 2
            plens = [oxn_p, oen_p]
            poffs = [0, oxn_p]
        t_ws = []
        for t in range(nhg):
            for j in range(n_parts):
                t_ws.append(_kron8(_eye(plens[j], 16, poffs[j] - 16 * t)))
        n_ws = [bf(_kron8(Wxn[po:po + pl_])) for po, pl_ in zip(poffs, plens)]
        consts = consts + t_ws + n_ws + [_tile8(bxn)]

        def nk_fn(*vals, _np=n_parts, _nh=nhg, _nscp=nscp):
            vals = list(vals)
            if _np == 1:
                parts = [vals.pop(0)]
            else:
                x2p_b = vals.pop(0)
                es_parts = [vals.pop(0) for _ in range(2 * _nscp)]
                wenh = [vals.pop(0) for _ in range(_nscp)]
                benp_b = vals.pop(0)
                xa = jnp.maximum(x2p_b, 0.0)
                es_m = sum(_d1(es_parts[2 * h] + es_parts[2 * h + 1],
                               wenh[h]) for h in range(_nscp))
                xb = jnp.maximum(es_m + benp_b, 0.0)
                parts = [xa, xb]
            tw = [vals.pop(0) for _ in range(_nh * _np)]
            nw = [vals.pop(0) for _ in range(_np)]
            bxn_b = vals.pop(0)
            tables = [sum(_dot(p_, tw[t * _np + j])
                          for j, p_ in enumerate(parts))
                      for t in range(_nh)]
            x2 = sum(_d1(p_, w_) for p_, w_ in zip(parts, nw)) + bxn_b
            return tables + [x2]

        nk_outs = _rowmap(nk_fn, n8, bnp, row_ins, consts,
                          [128] * nhg + [8 * oxn])
        tables = [tp.reshape(npad, 16) for tp in nk_outs[:nhg]]
        x2 = nk_outs[nhg]

        g_outs = _sc_gather(tables, src, dst)
        g_packed = [o.reshape(e8, 128) for o in g_outs]

        if ea_halves is None:
            ea_ins = [eap]
            wee_parts = [bf(_kron8(_pad_cols(Wee, 16)))]
            oe_prev = None
        else:
            ea_ins = ea_halves
            wee_parts = [bf(_kron8(_pad_cols(_p16r(Wee[:oe_prev]), 16)))]
            for t in range(len(ea_halves) - 1):
                rows = Wee[oe_prev + 16 * t:min(oe_prev + 16 * (t + 1), ie)]
                wee_parts.append(bf(_kron8(_pad_cols(_p16r(rows), 16))))
        bee_t = _tile8(jnp.pad(bee.reshape(-1), (0, 16 - oe)))
        bxe_halves = [
            _tile8(jnp.pad(bxe[16 * t:16 * (t + 1)],
                           (0, 16 * (t + 1) - min(16 * (t + 1), ox))))
            for t in range(nbh)]
        wx_st = []
        for s in range(nhg):
            for t in range(nbh):
                blkw = Wxe[16 * s:16 * (s + 1), 16 * t:16 * (t + 1)]
                wx_st.append(bf(_kron8(_pad_cols(_p16r(blkw), 16))))
        e_consts = wee_parts + [bee_t] + wx_st + bxe_halves
        n_ea = len(ea_ins)

        def ek_fn(*vals, _n_ea=n_ea, _nh=nhg, _nb=nbh):
            vals = list(vals)
            ea_parts = [vals.pop(0) for _ in range(_n_ea)]
            g_bs = [vals.pop(0) for _ in range(2 * _nh)]
            wee_p = [vals.pop(0) for _ in range(_n_ea)]
            bee_b = vals.pop(0)
            wx_b = [vals.pop(0) for _ in range(_nh * _nb)]
            bxe_b = [vals.pop(0) for _ in range(_nb)]
            ea2 = sum(_d1(ap, w_) for ap, w_ in zip(ea_parts, wee_p)) + bee_b
            a_ = jnp.maximum(ea2, 0.0)
            gsum = [g_bs[2 * t] + g_bs[2 * t + 1] for t in range(_nh)]
            outs = [a_]
            for t in range(_nb):
                nst = sum(_d1(gsum[s], wx_b[s * _nb + t])
                          for s in range(_nh)) + bxe_b[t]
                outs.append(jnp.maximum(nst, 0.0))
            return outs

        ek_outs = _rowmap(ek_fn, e8, be, ea_ins + g_packed, e_consts,
                          [128] * nsc)
        ea_halves = list(ek_outs)
        oe_prev = oe

        pays = [h.reshape(e, _WSP) for h in ea_halves]
        parts_sc = []
        q = 0
        while q < nsc:
            grp = pays[q:q + 2]
            outs = _sc_scatter(grp, dst, npad)
            parts_sc.extend(o.reshape(n8, 128) for o in outs)
            q += 2
        x_carry = ("pend", x2, parts_sc, Wen, ben, oe, ien, oxn, oen)

    (_, x2f, parts_f, Wen_f, ben_f, oe_f, ie_f, oxnf, oenf) = x_carry
    nscf = len(parts_f) // 2
    d5 = oxnf + oenf
    wenf_halves = [bf(_kron8(_p16r(Wen_f[:oe_f])))]
    for t in range(nscf - 1):
        wenf_halves.append(bf(_kron8(_p16r(
            Wen_f[oe_f + 16 * t:min(oe_f + 16 * (t + 1), ie_f)]))))
    benf_t = _tile8(ben_f)
    ka = _kron8(_eye(oxnf, d5))
    kb = _kron8(_eye(oenf, d5, oxnf))
    gsum_m = jnp.tile(_eye(d5, d5), (8, 1))
    kx = _kron8(_eye(d5, 2 * d5))
    ks = jnp.tile(_eye(d5, 2 * d5, d5), (1, 8))
    n8_valid = n // 8
    grid = (n8 // bnp,)

    def f1_body(*refs):
        x2_r = refs[0]
        es_rs = refs[1:1 + 2 * nscf]
        wenh = refs[1 + 2 * nscf:1 + 3 * nscf]
        bent_r, ka_r, kb_r = refs[1 + 3 * nscf:4 + 3 * nscf]
        o5_r, s_r = refs[4 + 3 * nscf:]
        xa = jnp.maximum(x2_r[...], 0.0)
        es_m = sum(_d1(es_rs[2 * h][...] + es_rs[2 * h + 1][...],
                       wenh[h][...]) for h in range(nscf))
        xb = jnp.maximum(es_m + bent_r[...], 0.0)
        x5 = _dot(xa, ka_r[...]) + _dot(xb, kb_r[...])
        o5_r[...] = x5
        rid = (pl.program_id(0) * bnp
               + jax.lax.broadcasted_iota(jnp.int32, (bnp, 1), 0))
        x5m = jnp.where(rid < n8_valid, x5, 0.0)

        @pl.when(pl.program_id(0) == 0)
        def _():
            s_r[...] = jnp.zeros_like(s_r)

        s_r[...] += jnp.sum(x5m, axis=0, keepdims=True)

    f1_consts = wenf_halves + [benf_t, ka, kb]
    o5, ssum = pl.pallas_call(
        f1_body, grid=grid,
        in_specs=([pl.BlockSpec((bnp, x2f.shape[1]), lambda g: (g, 0))]
                  + [pl.BlockSpec((bnp, 128), lambda g: (g, 0))
                     for _ in parts_f]
                  + [pl.BlockSpec(c.shape, lambda g: (0, 0))
                     for c in f1_consts]),
        out_specs=[pl.BlockSpec((bnp, 8 * d5), lambda g: (g, 0)),
                   pl.BlockSpec((1, 8 * d5), lambda g: (0, 0))],
        out_shape=[jax.ShapeDtypeStruct((n8, 8 * d5), F32),
                   jax.ShapeDtypeStruct((1, 8 * d5), F32)],
    )(x2f, *parts_f, *f1_consts)

    def f2_body(o5_r, s_r, gs_r, kx_r, ks_r, out_r):
        s5 = _dot(s_r[...], gs_r[...])
        out_r[...] = _dot(o5_r[...], kx_r[...]) + jnp.broadcast_to(
            _dot(s5, ks_r[...]), (bnp, 16 * d5))

    outp = pl.pallas_call(
        f2_body, grid=grid,
        in_specs=[pl.BlockSpec((bnp, 8 * d5), lambda g: (g, 0)),
                  pl.BlockSpec((1, 8 * d5), lambda g: (0, 0)),
                  pl.BlockSpec(gsum_m.shape, lambda g: (0, 0)),
                  pl.BlockSpec(kx.shape, lambda g: (0, 0)),
                  pl.BlockSpec(ks.shape, lambda g: (0, 0))],
        out_specs=pl.BlockSpec((bnp, 16 * d5), lambda g: (g, 0)),
        out_shape=jax.ShapeDtypeStruct((n8, 16 * d5), F32),
    )(o5, ssum, gsum_m, kx, ks)
    return outp.reshape(npad, 2 * d5)[:n]
